# Initial kernel scaffold; baseline (speedup 1.0000x reference)
import jax
import jax.numpy as jnp
from jax.experimental import pallas as pl


def kernel(x, edge_index, edge_attr, W, att, bias, bn_gamma, bn_beta):
    raise NotImplementedError("write your pallas kernel here")


# speedup vs baseline: 6.2706x; 6.2706x over previous
import functools

import jax
import jax.numpy as jnp
from jax import lax
from jax.experimental import pallas as pl
from jax.experimental.pallas import tpu as pltpu
from jax.experimental.pallas import tpu_sc as plsc

NC = 2
NS = 16
NW = NC * NS
CH = 128

BE = 1000


def _sp(x):
    return jnp.maximum(x, 0.0) + jnp.log1p(jnp.exp(-jnp.abs(x)))


def _sc_gather_x(x, idx_i, idx_j):
    E = idx_i.shape[0]
    N, DIM = x.shape
    nchunks = E // CH
    per_tile = -(-nchunks // NW)
    mesh = plsc.VectorSubcoreMesh(core_axis_name="c", subcore_axis_name="s")

    @functools.partial(
        pl.kernel, mesh=mesh,
        out_type=(jax.ShapeDtypeStruct((E, DIM), jnp.float32),
                  jax.ShapeDtypeStruct((E, DIM), jnp.float32)),
        scratch_types=[pltpu.VMEM((CH,), jnp.int32),
                       pltpu.VMEM((CH, DIM), jnp.float32)],
    )
    def k(x_hbm, ii_hbm, ij_hbm, gi_hbm, gj_hbm, idx_v, rows_v):
        wid = lax.axis_index("s") * NC + lax.axis_index("c")

        @pl.loop(0, per_tile)
        def _(t):
            kk = wid + NW * t

            @pl.when(kk < nchunks)
            def _():
                base = kk * CH
                pltpu.sync_copy(ii_hbm.at[pl.ds(base, CH)], idx_v)
                pltpu.sync_copy(x_hbm.at[idx_v], rows_v)
                pltpu.sync_copy(rows_v, gi_hbm.at[pl.ds(base, CH)])
                pltpu.sync_copy(ij_hbm.at[pl.ds(base, CH)], idx_v)
                pltpu.sync_copy(x_hbm.at[idx_v], rows_v)
                pltpu.sync_copy(rows_v, gj_hbm.at[pl.ds(base, CH)])

    return k(x, idx_i, idx_j)


def _sc_denom_scatter(ex16, idx_i, n_nodes):
    E = ex16.shape[0]
    nchunks = E // CH
    per_core = nchunks // NC
    per_tile = -(-per_core // NS)
    n_pad = -(-n_nodes // (8 * NS)) * 8 * NS
    rows_per_tile = n_pad // NS
    mesh = plsc.VectorSubcoreMesh(core_axis_name="c", subcore_axis_name="s")
    zrows = jnp.zeros((rows_per_tile, 16), jnp.float32)

    @functools.partial(
        pl.kernel, mesh=mesh,
        out_type=jax.ShapeDtypeStruct((NC, n_pad, 16), jnp.float32),
        scratch_types=[pltpu.VMEM((CH,), jnp.int32),
                       pltpu.VMEM((CH, 16), jnp.float32),
                       pltpu.VMEM_SHARED((n_pad, 16), jnp.float32)],
        compiler_params=pltpu.CompilerParams(use_tc_tiling_on_sc=False),
    )
    def k(ex_hbm, ii_hbm, z_hbm, den_hbm, idx_v, rows_v, acc_shared):
        c = lax.axis_index("c")
        s = lax.axis_index("s")
        pltpu.sync_copy(z_hbm, acc_shared.at[pl.ds(s * rows_per_tile, rows_per_tile)])
        plsc.subcore_barrier()

        @pl.loop(0, per_tile)
        def _(t):
            kk = s + NS * t

            @pl.when(kk < per_core)
            def _():
                base = (c * per_core + kk) * CH
                pltpu.sync_copy(ii_hbm.at[pl.ds(base, CH)], idx_v)
                pltpu.sync_copy(ex_hbm.at[pl.ds(base, CH)], rows_v)
                pltpu.sync_copy(rows_v, acc_shared.at[idx_v], add=True)

        plsc.subcore_barrier()
        sl = pl.ds(s * rows_per_tile, rows_per_tile)
        pltpu.sync_copy(acc_shared.at[sl], den_hbm.at[c].at[sl])

    return k(ex16, idx_i, zrows)


def _sc_denom_gather(den2, idx_i, idx_i2):
    E = idx_i.shape[0]
    nchunks = E // CH
    per_tile = -(-nchunks // NW)
    mesh = plsc.VectorSubcoreMesh(core_axis_name="c", subcore_axis_name="s")

    @functools.partial(
        pl.kernel, mesh=mesh,
        out_type=(jax.ShapeDtypeStruct((E, 16), jnp.float32),
                  jax.ShapeDtypeStruct((E, 16), jnp.float32)),
        scratch_types=[pltpu.VMEM((CH,), jnp.int32),
                       pltpu.VMEM((CH, 16), jnp.float32)],
        compiler_params=pltpu.CompilerParams(use_tc_tiling_on_sc=False),
    )
    def k(den_hbm, ii_hbm, ii2_hbm, dg0_hbm, dg1_hbm, idx_v, rows_v):
        wid = lax.axis_index("s") * NC + lax.axis_index("c")

        @pl.loop(0, per_tile)
        def _(t):
            kk = wid + NW * t

            @pl.when(kk < nchunks)
            def _():
                base = kk * CH
                pltpu.sync_copy(ii_hbm.at[pl.ds(base, CH)], idx_v)
                pltpu.sync_copy(den_hbm.at[idx_v], rows_v)
                pltpu.sync_copy(rows_v, dg0_hbm.at[pl.ds(base, CH)])
                pltpu.sync_copy(ii2_hbm.at[pl.ds(base, CH)], idx_v)
                pltpu.sync_copy(den_hbm.at[idx_v], rows_v)
                pltpu.sync_copy(rows_v, dg1_hbm.at[pl.ds(base, CH)])

    return k(den2, idx_i, idx_i2)


def _sc_aggregate(msgr, idx_i, n_nodes):
    E, DIM = msgr.shape
    nchunks = E // CH
    per_core = nchunks // NC
    per_tile = -(-per_core // NS)
    n_pad = -(-n_nodes // (8 * NS)) * 8 * NS
    rows_per_tile = n_pad // NS
    mesh = plsc.VectorSubcoreMesh(core_axis_name="c", subcore_axis_name="s")
    zrows = jnp.zeros((rows_per_tile, DIM), jnp.float32)

    @functools.partial(
        pl.kernel, mesh=mesh,
        out_type=jax.ShapeDtypeStruct((NC, n_pad, DIM), jnp.float32),
        scratch_types=[pltpu.VMEM((CH,), jnp.int32),
                       pltpu.VMEM((CH, DIM), jnp.float32),
                       pltpu.VMEM_SHARED((n_pad, DIM), jnp.float32)],
    )
    def k(m_hbm, ii_hbm, z_hbm, agg_hbm, idx_v, rows_v, acc_shared):
        c = lax.axis_index("c")
        s = lax.axis_index("s")
        pltpu.sync_copy(z_hbm, acc_shared.at[pl.ds(s * rows_per_tile, rows_per_tile)])
        plsc.subcore_barrier()

        @pl.loop(0, per_tile)
        def _(t):
            kk = s + NS * t

            @pl.when(kk < per_core)
            def _():
                base = (c * per_core + kk) * CH
                pltpu.sync_copy(ii_hbm.at[pl.ds(base, CH)], idx_v)
                pltpu.sync_copy(m_hbm.at[pl.ds(base, CH)], rows_v)
                pltpu.sync_copy(rows_v, acc_shared.at[idx_v], add=True)

        plsc.subcore_barrier()
        sl = pl.ds(s * rows_per_tile, rows_per_tile)
        pltpu.sync_copy(acc_shared.at[sl], agg_hbm.at[c].at[sl])

    return k(msgr, idx_i, zrows)


def _tc_edge(gxi, gxj, ea, W1, W2, A1, A2):
    E, DIM = ea.shape
    HD = W1.shape[1]
    H = A1.shape[1]
    nb = E // BE

    def body(gxi_ref, gxj_ref, ea_ref, w1_ref, w2_ref, a1_ref, a2_ref,
             oj_ref, al_ref, s_ref, ss_ref):
        t = jnp.dot(ea_ref[...], w2_ref[...], preferred_element_type=jnp.float32,
                    precision=lax.Precision.HIGHEST)
        ui = jnp.dot(gxi_ref[...], w1_ref[...], preferred_element_type=jnp.float32,
                     precision=lax.Precision.HIGHEST) + t
        uj = jnp.dot(gxj_ref[...], w1_ref[...], preferred_element_type=jnp.float32,
                     precision=lax.Precision.HIGHEST) + t
        oi = _sp(ui)
        oj = _sp(uj)
        oj_ref[...] = oj
        al = (jnp.dot(oi, a1_ref[...], preferred_element_type=jnp.float32,
                      precision=lax.Precision.HIGHEST)
              + jnp.dot(oj, a2_ref[...], preferred_element_type=jnp.float32,
                        precision=lax.Precision.HIGHEST))
        al = _sp(al)
        al_ref[...] = al
        s_ref[...] = al.sum(axis=0).reshape(1, 1, H)
        ss_ref[...] = (al * al).sum(axis=0).reshape(1, 1, H)

    return pl.pallas_call(
        body,
        grid=(nb,),
        in_specs=[
            pl.BlockSpec((BE, DIM), lambda i: (i, 0)),
            pl.BlockSpec((BE, DIM), lambda i: (i, 0)),
            pl.BlockSpec((BE, DIM), lambda i: (i, 0)),
            pl.BlockSpec((DIM, HD), lambda i: (0, 0)),
            pl.BlockSpec((DIM, HD), lambda i: (0, 0)),
            pl.BlockSpec((HD, H), lambda i: (0, 0)),
            pl.BlockSpec((HD, H), lambda i: (0, 0)),
        ],
        out_specs=[
            pl.BlockSpec((BE, HD), lambda i: (i, 0)),
            pl.BlockSpec((BE, H), lambda i: (i, 0)),
            pl.BlockSpec((1, 1, H), lambda i: (i, 0, 0)),
            pl.BlockSpec((1, 1, H), lambda i: (i, 0, 0)),
        ],
        out_shape=[
            jax.ShapeDtypeStruct((E, HD), jnp.float32),
            jax.ShapeDtypeStruct((E, H), jnp.float32),
            jax.ShapeDtypeStruct((nb, 1, H), jnp.float32),
            jax.ShapeDtypeStruct((nb, 1, H), jnp.float32),
        ],
    )(gxi, gxj, ea, W1, W2, A1, A2)


def _tc_bn_exp(alpha, params):
    E, H = alpha.shape
    nb = E // BE

    def body(al_ref, p_ref, ex_ref):
        al = al_ref[...]
        mean = p_ref[0:1, :]
        rstdg = p_ref[1:2, :]
        beta = p_ref[2:3, :]
        z = (al - mean) * rstdg + beta
        ex = jnp.exp(_sp(z))
        ex_ref[...] = jnp.concatenate(
            [ex, jnp.zeros((BE, 16 - H), jnp.float32)], axis=1)

    return pl.pallas_call(
        body,
        grid=(nb,),
        in_specs=[
            pl.BlockSpec((BE, H), lambda i: (i, 0)),
            pl.BlockSpec((8, H), lambda i: (0, 0)),
        ],
        out_specs=pl.BlockSpec((BE, 16), lambda i: (i, 0)),
        out_shape=jax.ShapeDtypeStruct((E, 16), jnp.float32),
    )(alpha, params)


def _tc_combine(oj, ex16, dg0, dg1):
    E, HD = oj.shape
    H = 4
    DIM = HD // H
    nb = E // BE

    def body(oj_ref, ex_ref, d0_ref, d1_ref, m_ref):
        w = ex_ref[:, :H] / (d0_ref[:, :H] + d1_ref[:, :H] + 1e-16) * 0.25
        oj = oj_ref[...]
        acc = oj[:, 0:DIM] * w[:, 0:1]
        for h in range(1, H):
            acc = acc + oj[:, h * DIM:(h + 1) * DIM] * w[:, h:h + 1]
        m_ref[...] = acc

    return pl.pallas_call(
        body,
        grid=(nb,),
        in_specs=[
            pl.BlockSpec((BE, HD), lambda i: (i, 0)),
            pl.BlockSpec((BE, 16), lambda i: (i, 0)),
            pl.BlockSpec((BE, 16), lambda i: (i, 0)),
            pl.BlockSpec((BE, 16), lambda i: (i, 0)),
        ],
        out_specs=pl.BlockSpec((BE, DIM), lambda i: (i, 0)),
        out_shape=jax.ShapeDtypeStruct((E, DIM), jnp.float32),
    )(oj, ex16, dg0, dg1)


def _tc_finalize(p0, p1, biasb):
    N, DIM = p0.shape
    BN = 1000
    nb = N // BN

    def body(a_ref, b_ref, bias_ref, o_ref):
        o_ref[...] = a_ref[...] + b_ref[...] + bias_ref[0:1, :]

    return pl.pallas_call(
        body,
        grid=(nb,),
        in_specs=[
            pl.BlockSpec((BN, DIM), lambda i: (i, 0)),
            pl.BlockSpec((BN, DIM), lambda i: (i, 0)),
            pl.BlockSpec((8, DIM), lambda i: (0, 0)),
        ],
        out_specs=pl.BlockSpec((BN, DIM), lambda i: (i, 0)),
        out_shape=jax.ShapeDtypeStruct((N, DIM), jnp.float32),
    )(p0, p1, biasb)


def kernel(x, edge_index, edge_attr, W, att, bias, bn_gamma, bn_beta):
    N, DIM = x.shape
    E = edge_attr.shape[0]
    H = att.shape[1]

    idx_i = edge_index[0]
    idx_j = edge_index[1]
    W1 = W[:DIM]
    W2 = W[DIM:]
    eye = jnp.eye(H, dtype=jnp.float32)
    A1 = (att[0, :, :DIM][:, :, None] * eye[:, None, :]).reshape(H * DIM, H)
    A2 = (att[0, :, DIM:][:, :, None] * eye[:, None, :]).reshape(H * DIM, H)

    gxi, gxj = _sc_gather_x(x, idx_i, idx_j)
    oj, alpha, s_part, ss_part = _tc_edge(gxi, gxj, edge_attr, W1, W2, A1, A2)

    s = s_part.sum(axis=(0, 1))
    ss = ss_part.sum(axis=(0, 1))
    mean = s / E
    var = ss / E - mean * mean
    rstdg = bn_gamma / jnp.sqrt(var + 1e-5)
    params = jnp.zeros((8, H), jnp.float32)
    params = params.at[0].set(mean).at[1].set(rstdg).at[2].set(bn_beta)

    ex16 = _tc_bn_exp(alpha, params)
    den = _sc_denom_scatter(ex16, idx_i, N)
    n_pad = den.shape[1]
    den2 = den.reshape(NC * n_pad, 16)
    dg0, dg1 = _sc_denom_gather(den2, idx_i, idx_i + n_pad)
    msgr = _tc_combine(oj, ex16, dg0, dg1)
    agg = _sc_aggregate(msgr, idx_i, N)
    biasb = jnp.broadcast_to(bias, (8, DIM))
    return _tc_finalize(agg[0, :N], agg[1, :N], biasb)


# speedup vs baseline: 13.5239x; 2.1567x over previous
import functools

import jax
import jax.numpy as jnp
from jax import lax
from jax.experimental import pallas as pl
from jax.experimental.pallas import tpu as pltpu
from jax.experimental.pallas import tpu_sc as plsc

NC = 2
NS = 16
NW = NC * NS
CH = 128

BE = 1000


def _sp(x):
    return jnp.maximum(x, 0.0) + jnp.log1p(jnp.exp(-jnp.abs(x)))


def _sc_gather_x(x, idx_i, idx_j):
    E = idx_i.shape[0]
    N, DIM = x.shape
    nchunks = E // CH
    per_tile = -(-nchunks // NW)
    mesh = plsc.VectorSubcoreMesh(core_axis_name="c", subcore_axis_name="s")

    @functools.partial(
        pl.kernel, mesh=mesh,
        out_type=(jax.ShapeDtypeStruct((E, DIM), jnp.float32),
                  jax.ShapeDtypeStruct((E, DIM), jnp.float32)),
        scratch_types=[pltpu.VMEM((CH,), jnp.int32),
                       pltpu.VMEM((CH, DIM), jnp.float32)],
    )
    def k(x_hbm, ii_hbm, ij_hbm, gi_hbm, gj_hbm, idx_v, rows_v):
        wid = lax.axis_index("s") * NC + lax.axis_index("c")

        @pl.loop(0, per_tile)
        def _(t):
            kk = wid + NW * t

            @pl.when(kk < nchunks)
            def _():
                base = kk * CH
                pltpu.sync_copy(ii_hbm.at[pl.ds(base, CH)], idx_v)
                pltpu.sync_copy(x_hbm.at[idx_v], rows_v)
                pltpu.sync_copy(rows_v, gi_hbm.at[pl.ds(base, CH)])
                pltpu.sync_copy(ij_hbm.at[pl.ds(base, CH)], idx_v)
                pltpu.sync_copy(x_hbm.at[idx_v], rows_v)
                pltpu.sync_copy(rows_v, gj_hbm.at[pl.ds(base, CH)])

    return k(x, idx_i, idx_j)


def _sc_denom_scatter(ex16, idx_i, n_nodes):
    E = ex16.shape[0]
    nchunks = E // CH
    per_core = nchunks // NC
    per_tile = -(-per_core // NS)
    n_pad = -(-n_nodes // (8 * NS)) * 8 * NS
    rows_per_tile = n_pad // NS
    mesh = plsc.VectorSubcoreMesh(core_axis_name="c", subcore_axis_name="s")
    zrows = jnp.zeros((rows_per_tile, 16), jnp.float32)

    @functools.partial(
        pl.kernel, mesh=mesh,
        out_type=jax.ShapeDtypeStruct((NC, n_pad, 16), jnp.float32),
        scratch_types=[pltpu.VMEM((CH,), jnp.int32),
                       pltpu.VMEM((CH, 16), jnp.float32),
                       pltpu.VMEM_SHARED((n_pad, 16), jnp.float32)],
        compiler_params=pltpu.CompilerParams(use_tc_tiling_on_sc=False),
    )
    def k(ex_hbm, ii_hbm, z_hbm, den_hbm, idx_v, rows_v, acc_shared):
        c = lax.axis_index("c")
        s = lax.axis_index("s")
        pltpu.sync_copy(z_hbm, acc_shared.at[pl.ds(s * rows_per_tile, rows_per_tile)])
        plsc.subcore_barrier()

        @pl.loop(0, per_tile)
        def _(t):
            kk = s + NS * t

            @pl.when(kk < per_core)
            def _():
                base = (c * per_core + kk) * CH
                pltpu.sync_copy(ii_hbm.at[pl.ds(base, CH)], idx_v)
                pltpu.sync_copy(ex_hbm.at[pl.ds(base, CH)], rows_v)
                pltpu.sync_copy(rows_v, acc_shared.at[idx_v], add=True)

        plsc.subcore_barrier()
        sl = pl.ds(s * rows_per_tile, rows_per_tile)
        pltpu.sync_copy(acc_shared.at[sl], den_hbm.at[c].at[sl])

    return k(ex16, idx_i, zrows)


def _sc_denom_gather(den2, idx_i, idx_i2):
    E = idx_i.shape[0]
    nchunks = E // CH
    per_tile = -(-nchunks // NW)
    mesh = plsc.VectorSubcoreMesh(core_axis_name="c", subcore_axis_name="s")

    @functools.partial(
        pl.kernel, mesh=mesh,
        out_type=(jax.ShapeDtypeStruct((E, 16), jnp.float32),
                  jax.ShapeDtypeStruct((E, 16), jnp.float32)),
        scratch_types=[pltpu.VMEM((CH,), jnp.int32),
                       pltpu.VMEM((CH, 16), jnp.float32)],
        compiler_params=pltpu.CompilerParams(use_tc_tiling_on_sc=False),
    )
    def k(den_hbm, ii_hbm, ii2_hbm, dg0_hbm, dg1_hbm, idx_v, rows_v):
        wid = lax.axis_index("s") * NC + lax.axis_index("c")

        @pl.loop(0, per_tile)
        def _(t):
            kk = wid + NW * t

            @pl.when(kk < nchunks)
            def _():
                base = kk * CH
                pltpu.sync_copy(ii_hbm.at[pl.ds(base, CH)], idx_v)
                pltpu.sync_copy(den_hbm.at[idx_v], rows_v)
                pltpu.sync_copy(rows_v, dg0_hbm.at[pl.ds(base, CH)])
                pltpu.sync_copy(ii2_hbm.at[pl.ds(base, CH)], idx_v)
                pltpu.sync_copy(den_hbm.at[idx_v], rows_v)
                pltpu.sync_copy(rows_v, dg1_hbm.at[pl.ds(base, CH)])

    return k(den2, idx_i, idx_i2)


def _sc_aggregate(msgr, idx_i, n_nodes):
    E, DIM = msgr.shape
    nchunks = E // CH
    per_core = nchunks // NC
    per_tile = -(-per_core // NS)
    n_pad = -(-n_nodes // (8 * NS)) * 8 * NS
    rows_per_tile = n_pad // NS
    mesh = plsc.VectorSubcoreMesh(core_axis_name="c", subcore_axis_name="s")
    zrows = jnp.zeros((rows_per_tile, DIM), jnp.float32)

    @functools.partial(
        pl.kernel, mesh=mesh,
        out_type=jax.ShapeDtypeStruct((NC, n_pad, DIM), jnp.float32),
        scratch_types=[pltpu.VMEM((CH,), jnp.int32),
                       pltpu.VMEM((CH, DIM), jnp.float32),
                       pltpu.VMEM_SHARED((n_pad, DIM), jnp.float32)],
    )
    def k(m_hbm, ii_hbm, z_hbm, agg_hbm, idx_v, rows_v, acc_shared):
        c = lax.axis_index("c")
        s = lax.axis_index("s")
        pltpu.sync_copy(z_hbm, acc_shared.at[pl.ds(s * rows_per_tile, rows_per_tile)])
        plsc.subcore_barrier()

        @pl.loop(0, per_tile)
        def _(t):
            kk = s + NS * t

            @pl.when(kk < per_core)
            def _():
                base = (c * per_core + kk) * CH
                pltpu.sync_copy(ii_hbm.at[pl.ds(base, CH)], idx_v)
                pltpu.sync_copy(m_hbm.at[pl.ds(base, CH)], rows_v)
                pltpu.sync_copy(rows_v, acc_shared.at[idx_v], add=True)

        plsc.subcore_barrier()
        sl = pl.ds(s * rows_per_tile, rows_per_tile)
        pltpu.sync_copy(acc_shared.at[sl], agg_hbm.at[c].at[sl])

    return k(msgr, idx_i, zrows)


def _tc_edge(gxi, gxj, ea, Wb, A1b, A2b):
    E, DIM = ea.shape
    HD = Wb.shape[1]
    H = A1b.shape[1]
    nb = E // BE

    def body(gxi_ref, gxj_ref, ea_ref, w_ref, a1_ref, a2_ref,
             oj_ref, al_ref, s_ref, ss_ref):
        eab = ea_ref[...].astype(jnp.bfloat16)
        ci = jnp.concatenate([gxi_ref[...].astype(jnp.bfloat16), eab], axis=1)
        cj = jnp.concatenate([gxj_ref[...].astype(jnp.bfloat16), eab], axis=1)
        w = w_ref[...]
        ui = jnp.dot(ci, w, preferred_element_type=jnp.float32)
        uj = jnp.dot(cj, w, preferred_element_type=jnp.float32)
        oi = _sp(ui)
        oj = _sp(uj)
        oj_ref[...] = oj
        al = (jnp.dot(oi.astype(jnp.bfloat16), a1_ref[...],
                      preferred_element_type=jnp.float32)
              + jnp.dot(oj.astype(jnp.bfloat16), a2_ref[...],
                        preferred_element_type=jnp.float32))
        al = _sp(al)
        al_ref[...] = al
        s_ref[...] = al.sum(axis=0).reshape(1, 1, H)
        ss_ref[...] = (al * al).sum(axis=0).reshape(1, 1, H)

    return pl.pallas_call(
        body,
        grid=(nb,),
        in_specs=[
            pl.BlockSpec((BE, DIM), lambda i: (i, 0)),
            pl.BlockSpec((BE, DIM), lambda i: (i, 0)),
            pl.BlockSpec((BE, DIM), lambda i: (i, 0)),
            pl.BlockSpec((2 * DIM, HD), lambda i: (0, 0)),
            pl.BlockSpec((HD, H), lambda i: (0, 0)),
            pl.BlockSpec((HD, H), lambda i: (0, 0)),
        ],
        out_specs=[
            pl.BlockSpec((BE, HD), lambda i: (i, 0)),
            pl.BlockSpec((BE, H), lambda i: (i, 0)),
            pl.BlockSpec((1, 1, H), lambda i: (i, 0, 0)),
            pl.BlockSpec((1, 1, H), lambda i: (i, 0, 0)),
        ],
        out_shape=[
            jax.ShapeDtypeStruct((E, HD), jnp.float32),
            jax.ShapeDtypeStruct((E, H), jnp.float32),
            jax.ShapeDtypeStruct((nb, 1, H), jnp.float32),
            jax.ShapeDtypeStruct((nb, 1, H), jnp.float32),
        ],
    )(gxi, gxj, ea, Wb, A1b, A2b)


def _tc_bn_exp(alpha, params):
    E, H = alpha.shape
    nb = E // BE

    def body(al_ref, p_ref, ex_ref):
        al = al_ref[...]
        mean = p_ref[0:1, :]
        rstdg = p_ref[1:2, :]
        beta = p_ref[2:3, :]
        z = (al - mean) * rstdg + beta
        ex = jnp.exp(_sp(z))
        ex_ref[...] = jnp.concatenate(
            [ex, jnp.zeros((BE, 16 - H), jnp.float32)], axis=1)

    return pl.pallas_call(
        body,
        grid=(nb,),
        in_specs=[
            pl.BlockSpec((BE, H), lambda i: (i, 0)),
            pl.BlockSpec((8, H), lambda i: (0, 0)),
        ],
        out_specs=pl.BlockSpec((BE, 16), lambda i: (i, 0)),
        out_shape=jax.ShapeDtypeStruct((E, 16), jnp.float32),
    )(alpha, params)


def _tc_combine(oj, ex16, dg0, dg1):
    E, HD = oj.shape
    H = 4
    DIM = HD // H
    nb = E // BE

    def body(oj_ref, ex_ref, d0_ref, d1_ref, m_ref):
        w = ex_ref[:, :H] / (d0_ref[:, :H] + d1_ref[:, :H] + 1e-16) * 0.25
        oj = oj_ref[...]
        acc = oj[:, 0:DIM] * w[:, 0:1]
        for h in range(1, H):
            acc = acc + oj[:, h * DIM:(h + 1) * DIM] * w[:, h:h + 1]
        m_ref[...] = acc

    return pl.pallas_call(
        body,
        grid=(nb,),
        in_specs=[
            pl.BlockSpec((BE, HD), lambda i: (i, 0)),
            pl.BlockSpec((BE, 16), lambda i: (i, 0)),
            pl.BlockSpec((BE, 16), lambda i: (i, 0)),
            pl.BlockSpec((BE, 16), lambda i: (i, 0)),
        ],
        out_specs=pl.BlockSpec((BE, DIM), lambda i: (i, 0)),
        out_shape=jax.ShapeDtypeStruct((E, DIM), jnp.float32),
    )(oj, ex16, dg0, dg1)


def _tc_finalize(p0, p1, biasb):
    N, DIM = p0.shape
    BN = 1000
    nb = N // BN

    def body(a_ref, b_ref, bias_ref, o_ref):
        o_ref[...] = a_ref[...] + b_ref[...] + bias_ref[0:1, :]

    return pl.pallas_call(
        body,
        grid=(nb,),
        in_specs=[
            pl.BlockSpec((BN, DIM), lambda i: (i, 0)),
            pl.BlockSpec((BN, DIM), lambda i: (i, 0)),
            pl.BlockSpec((8, DIM), lambda i: (0, 0)),
        ],
        out_specs=pl.BlockSpec((BN, DIM), lambda i: (i, 0)),
        out_shape=jax.ShapeDtypeStruct((N, DIM), jnp.float32),
    )(p0, p1, biasb)


def kernel(x, edge_index, edge_attr, W, att, bias, bn_gamma, bn_beta):
    N, DIM = x.shape
    E = edge_attr.shape[0]
    H = att.shape[1]

    idx_i = edge_index[0]
    idx_j = edge_index[1]
    eye = jnp.eye(H, dtype=jnp.float32)
    A1 = (att[0, :, :DIM][:, :, None] * eye[:, None, :]).reshape(H * DIM, H)
    A2 = (att[0, :, DIM:][:, :, None] * eye[:, None, :]).reshape(H * DIM, H)
    Wb = W.astype(jnp.bfloat16)
    A1b = A1.astype(jnp.bfloat16)
    A2b = A2.astype(jnp.bfloat16)

    gxi, gxj = _sc_gather_x(x, idx_i, idx_j)
    oj, alpha, s_part, ss_part = _tc_edge(gxi, gxj, edge_attr, Wb, A1b, A2b)

    s = s_part.sum(axis=(0, 1))
    ss = ss_part.sum(axis=(0, 1))
    mean = s / E
    var = ss / E - mean * mean
    rstdg = bn_gamma / jnp.sqrt(var + 1e-5)
    params = jnp.zeros((8, H), jnp.float32)
    params = params.at[0].set(mean).at[1].set(rstdg).at[2].set(bn_beta)

    ex16 = _tc_bn_exp(alpha, params)
    den = _sc_denom_scatter(ex16, idx_i, N)
    n_pad = den.shape[1]
    den2 = den.reshape(NC * n_pad, 16)
    dg0, dg1 = _sc_denom_gather(den2, idx_i, idx_i + n_pad)
    msgr = _tc_combine(oj, ex16, dg0, dg1)
    agg = _sc_aggregate(msgr, idx_i, N)
    biasb = jnp.broadcast_to(bias, (8, DIM))
    return _tc_finalize(agg[0, :N], agg[1, :N], biasb)


# speedup vs baseline: 14.7424x; 1.0901x over previous
import functools

import jax
import jax.numpy as jnp
from jax import lax
from jax.experimental import pallas as pl
from jax.experimental.pallas import tpu as pltpu
from jax.experimental.pallas import tpu_sc as plsc

NC = 2
NS = 16
NW = NC * NS
CH = 128

BE = 1000


def _sp(x):
    return jnp.maximum(x, 0.0) + jnp.log1p(jnp.exp(-jnp.abs(x)))


def _sc_gather_x(x, idx_i, idx_j):
    E = idx_i.shape[0]
    N, DIM = x.shape
    nchunks = E // CH
    per_tile = -(-nchunks // NW)
    mesh = plsc.VectorSubcoreMesh(core_axis_name="c", subcore_axis_name="s")

    @functools.partial(
        pl.kernel, mesh=mesh,
        out_type=(jax.ShapeDtypeStruct((E, DIM), x.dtype),
                  jax.ShapeDtypeStruct((E, DIM), x.dtype)),
        scratch_types=[pltpu.VMEM((CH,), jnp.int32),
                       pltpu.VMEM((CH, DIM), x.dtype)],
    )
    def k(x_hbm, ii_hbm, ij_hbm, gi_hbm, gj_hbm, idx_v, rows_v):
        wid = lax.axis_index("s") * NC + lax.axis_index("c")

        @pl.loop(0, per_tile)
        def _(t):
            kk = wid + NW * t

            @pl.when(kk < nchunks)
            def _():
                base = kk * CH
                pltpu.sync_copy(ii_hbm.at[pl.ds(base, CH)], idx_v)
                pltpu.sync_copy(x_hbm.at[idx_v], rows_v)
                pltpu.sync_copy(rows_v, gi_hbm.at[pl.ds(base, CH)])
                pltpu.sync_copy(ij_hbm.at[pl.ds(base, CH)], idx_v)
                pltpu.sync_copy(x_hbm.at[idx_v], rows_v)
                pltpu.sync_copy(rows_v, gj_hbm.at[pl.ds(base, CH)])

    return k(x, idx_i, idx_j)


def _sc_denom_scatter(ex16, idx_i, n_nodes):
    E = ex16.shape[0]
    nchunks = E // CH
    per_core = nchunks // NC
    per_tile = -(-per_core // NS)
    n_pad = -(-n_nodes // (8 * NS)) * 8 * NS
    rows_per_tile = n_pad // NS
    mesh = plsc.VectorSubcoreMesh(core_axis_name="c", subcore_axis_name="s")
    zrows = jnp.zeros((rows_per_tile, 16), jnp.float32)

    @functools.partial(
        pl.kernel, mesh=mesh,
        out_type=jax.ShapeDtypeStruct((NC, n_pad, 16), jnp.float32),
        scratch_types=[pltpu.VMEM((CH,), jnp.int32),
                       pltpu.VMEM((CH, 16), jnp.float32),
                       pltpu.VMEM_SHARED((n_pad, 16), jnp.float32)],
        compiler_params=pltpu.CompilerParams(use_tc_tiling_on_sc=False),
    )
    def k(ex_hbm, ii_hbm, z_hbm, den_hbm, idx_v, rows_v, acc_shared):
        c = lax.axis_index("c")
        s = lax.axis_index("s")
        pltpu.sync_copy(z_hbm, acc_shared.at[pl.ds(s * rows_per_tile, rows_per_tile)])
        plsc.subcore_barrier()

        @pl.loop(0, per_tile)
        def _(t):
            kk = s + NS * t

            @pl.when(kk < per_core)
            def _():
                base = (c * per_core + kk) * CH
                pltpu.sync_copy(ii_hbm.at[pl.ds(base, CH)], idx_v)
                pltpu.sync_copy(ex_hbm.at[pl.ds(base, CH)], rows_v)
                pltpu.sync_copy(rows_v, acc_shared.at[idx_v], add=True)

        plsc.subcore_barrier()
        sl = pl.ds(s * rows_per_tile, rows_per_tile)
        pltpu.sync_copy(acc_shared.at[sl], den_hbm.at[c].at[sl])

    return k(ex16, idx_i, zrows)


def _sc_denom_gather(den, idx_i):
    E = idx_i.shape[0]
    nchunks = E // CH
    per_tile = -(-nchunks // NW)
    mesh = plsc.VectorSubcoreMesh(core_axis_name="c", subcore_axis_name="s")

    @functools.partial(
        pl.kernel, mesh=mesh,
        out_type=jax.ShapeDtypeStruct((E, 16), jnp.float32),
        scratch_types=[pltpu.VMEM((CH,), jnp.int32),
                       pltpu.VMEM((CH, 16), jnp.float32)],
        compiler_params=pltpu.CompilerParams(use_tc_tiling_on_sc=False),
    )
    def k(den_hbm, ii_hbm, dg_hbm, idx_v, rows_v):
        wid = lax.axis_index("s") * NC + lax.axis_index("c")

        @pl.loop(0, per_tile)
        def _(t):
            kk = wid + NW * t

            @pl.when(kk < nchunks)
            def _():
                base = kk * CH
                pltpu.sync_copy(ii_hbm.at[pl.ds(base, CH)], idx_v)
                pltpu.sync_copy(den_hbm.at[idx_v], rows_v)
                pltpu.sync_copy(rows_v, dg_hbm.at[pl.ds(base, CH)])

    return k(den, idx_i)


def _sc_aggregate(msgr, idx_i, n_nodes):
    E, DIM = msgr.shape
    nchunks = E // CH
    per_core = nchunks // NC
    per_tile = -(-per_core // NS)
    n_pad = -(-n_nodes // (8 * NS)) * 8 * NS
    rows_per_tile = n_pad // NS
    mesh = plsc.VectorSubcoreMesh(core_axis_name="c", subcore_axis_name="s")
    zrows = jnp.zeros((rows_per_tile, DIM), jnp.float32)

    @functools.partial(
        pl.kernel, mesh=mesh,
        out_type=jax.ShapeDtypeStruct((NC, n_pad, DIM), jnp.float32),
        scratch_types=[pltpu.VMEM((CH,), jnp.int32),
                       pltpu.VMEM((CH, DIM), jnp.float32),
                       pltpu.VMEM_SHARED((n_pad, DIM), jnp.float32)],
    )
    def k(m_hbm, ii_hbm, z_hbm, agg_hbm, idx_v, rows_v, acc_shared):
        c = lax.axis_index("c")
        s = lax.axis_index("s")
        pltpu.sync_copy(z_hbm, acc_shared.at[pl.ds(s * rows_per_tile, rows_per_tile)])
        plsc.subcore_barrier()

        @pl.loop(0, per_tile)
        def _(t):
            kk = s + NS * t

            @pl.when(kk < per_core)
            def _():
                base = (c * per_core + kk) * CH
                pltpu.sync_copy(ii_hbm.at[pl.ds(base, CH)], idx_v)
                pltpu.sync_copy(m_hbm.at[pl.ds(base, CH)], rows_v)
                pltpu.sync_copy(rows_v, acc_shared.at[idx_v], add=True)

        plsc.subcore_barrier()
        sl = pl.ds(s * rows_per_tile, rows_per_tile)
        pltpu.sync_copy(acc_shared.at[sl], agg_hbm.at[c].at[sl])

    return k(msgr, idx_i, zrows)


def _tc_edge(gxi, gxj, ea, Wb, A1b, A2b):
    E, DIM = ea.shape
    HD = Wb.shape[1]
    H = A1b.shape[1]
    nb = E // BE

    def body(gxi_ref, gxj_ref, ea_ref, w_ref, a1_ref, a2_ref,
             oj_ref, al_ref, s_ref, ss_ref):
        eab = ea_ref[...].astype(jnp.bfloat16)
        ci = jnp.concatenate([gxi_ref[...].astype(jnp.bfloat16), eab], axis=1)
        cj = jnp.concatenate([gxj_ref[...].astype(jnp.bfloat16), eab], axis=1)
        w = w_ref[...]
        ui = jnp.dot(ci, w, preferred_element_type=jnp.float32)
        uj = jnp.dot(cj, w, preferred_element_type=jnp.float32)
        oi = _sp(ui).astype(jnp.bfloat16)
        oj = _sp(uj).astype(jnp.bfloat16)
        oj_ref[...] = oj
        al = (jnp.dot(oi, a1_ref[...], preferred_element_type=jnp.float32)
              + jnp.dot(oj, a2_ref[...], preferred_element_type=jnp.float32))
        al = _sp(al)
        al_ref[...] = al
        s_ref[...] = al.sum(axis=0).reshape(1, 1, H)
        ss_ref[...] = (al * al).sum(axis=0).reshape(1, 1, H)

    return pl.pallas_call(
        body,
        grid=(nb,),
        in_specs=[
            pl.BlockSpec((BE, DIM), lambda i: (i, 0)),
            pl.BlockSpec((BE, DIM), lambda i: (i, 0)),
            pl.BlockSpec((BE, DIM), lambda i: (i, 0)),
            pl.BlockSpec((2 * DIM, HD), lambda i: (0, 0)),
            pl.BlockSpec((HD, H), lambda i: (0, 0)),
            pl.BlockSpec((HD, H), lambda i: (0, 0)),
        ],
        out_specs=[
            pl.BlockSpec((BE, HD), lambda i: (i, 0)),
            pl.BlockSpec((BE, H), lambda i: (i, 0)),
            pl.BlockSpec((1, 1, H), lambda i: (i, 0, 0)),
            pl.BlockSpec((1, 1, H), lambda i: (i, 0, 0)),
        ],
        out_shape=[
            jax.ShapeDtypeStruct((E, HD), jnp.bfloat16),
            jax.ShapeDtypeStruct((E, H), jnp.float32),
            jax.ShapeDtypeStruct((nb, 1, H), jnp.float32),
            jax.ShapeDtypeStruct((nb, 1, H), jnp.float32),
        ],
    )(gxi, gxj, ea, Wb, A1b, A2b)


def _tc_bn_exp(alpha, params):
    E, H = alpha.shape
    nb = E // BE

    def body(al_ref, p_ref, ex_ref):
        al = al_ref[...]
        mean = p_ref[0:1, :]
        rstdg = p_ref[1:2, :]
        beta = p_ref[2:3, :]
        z = (al - mean) * rstdg + beta
        ex = jnp.exp(_sp(z))
        ex_ref[...] = jnp.concatenate(
            [ex, jnp.zeros((BE, 16 - H), jnp.float32)], axis=1)

    return pl.pallas_call(
        body,
        grid=(nb,),
        in_specs=[
            pl.BlockSpec((BE, H), lambda i: (i, 0)),
            pl.BlockSpec((8, H), lambda i: (0, 0)),
        ],
        out_specs=pl.BlockSpec((BE, 16), lambda i: (i, 0)),
        out_shape=jax.ShapeDtypeStruct((E, 16), jnp.float32),
    )(alpha, params)


def _tc_densum(den):
    NP = den.shape[1]
    BN = NP // 8

    def body(d_ref, o_ref):
        o_ref[...] = d_ref[0] + d_ref[1]

    return pl.pallas_call(
        body,
        grid=(8,),
        in_specs=[pl.BlockSpec((2, BN, 16), lambda i: (0, i, 0))],
        out_specs=pl.BlockSpec((BN, 16), lambda i: (i, 0)),
        out_shape=jax.ShapeDtypeStruct((NP, 16), jnp.float32),
    )(den)


def _tc_combine(oj, ex16, dg):
    E, HD = oj.shape
    H = 4
    DIM = HD // H
    nb = E // BE

    def body(oj_ref, ex_ref, d_ref, m_ref):
        w = ex_ref[:, :H] / (d_ref[:, :H] + 1e-16) * 0.25
        oj = oj_ref[...].astype(jnp.float32)
        acc = oj[:, 0:DIM] * w[:, 0:1]
        for h in range(1, H):
            acc = acc + oj[:, h * DIM:(h + 1) * DIM] * w[:, h:h + 1]
        m_ref[...] = acc

    return pl.pallas_call(
        body,
        grid=(nb,),
        in_specs=[
            pl.BlockSpec((BE, HD), lambda i: (i, 0)),
            pl.BlockSpec((BE, 16), lambda i: (i, 0)),
            pl.BlockSpec((BE, 16), lambda i: (i, 0)),
        ],
        out_specs=pl.BlockSpec((BE, DIM), lambda i: (i, 0)),
        out_shape=jax.ShapeDtypeStruct((E, DIM), jnp.float32),
    )(oj, ex16, dg)


def _tc_finalize(p0, p1, biasb):
    N, DIM = p0.shape
    BN = 1000
    nb = N // BN

    def body(a_ref, b_ref, bias_ref, o_ref):
        o_ref[...] = a_ref[...] + b_ref[...] + bias_ref[0:1, :]

    return pl.pallas_call(
        body,
        grid=(nb,),
        in_specs=[
            pl.BlockSpec((BN, DIM), lambda i: (i, 0)),
            pl.BlockSpec((BN, DIM), lambda i: (i, 0)),
            pl.BlockSpec((8, DIM), lambda i: (0, 0)),
        ],
        out_specs=pl.BlockSpec((BN, DIM), lambda i: (i, 0)),
        out_shape=jax.ShapeDtypeStruct((N, DIM), jnp.float32),
    )(p0, p1, biasb)


def kernel(x, edge_index, edge_attr, W, att, bias, bn_gamma, bn_beta):
    N, DIM = x.shape
    E = edge_attr.shape[0]
    H = att.shape[1]

    idx_i = edge_index[0]
    idx_j = edge_index[1]
    eye = jnp.eye(H, dtype=jnp.float32)
    A1 = (att[0, :, :DIM][:, :, None] * eye[:, None, :]).reshape(H * DIM, H)
    A2 = (att[0, :, DIM:][:, :, None] * eye[:, None, :]).reshape(H * DIM, H)
    Wb = W.astype(jnp.bfloat16)
    A1b = A1.astype(jnp.bfloat16)
    A2b = A2.astype(jnp.bfloat16)

    gxi, gxj = _sc_gather_x(x, idx_i, idx_j)
    oj, alpha, s_part, ss_part = _tc_edge(gxi, gxj, edge_attr, Wb, A1b, A2b)

    s = s_part.sum(axis=(0, 1))
    ss = ss_part.sum(axis=(0, 1))
    mean = s / E
    var = ss / E - mean * mean
    rstdg = bn_gamma / jnp.sqrt(var + 1e-5)
    params = jnp.zeros((8, H), jnp.float32)
    params = params.at[0].set(mean).at[1].set(rstdg).at[2].set(bn_beta)

    ex16 = _tc_bn_exp(alpha, params)
    den = _sc_denom_scatter(ex16, idx_i, N)
    dg = _sc_denom_gather(_tc_densum(den), idx_i)
    msgr = _tc_combine(oj, ex16, dg)
    agg = _sc_aggregate(msgr, idx_i, N)
    biasb = jnp.broadcast_to(bias, (8, DIM))
    return _tc_finalize(agg[0, :N], agg[1, :N], biasb)


# speedup vs baseline: 14.7484x; 1.0004x over previous
import functools

import jax
import jax.numpy as jnp
from jax import lax
from jax.experimental import pallas as pl
from jax.experimental.pallas import tpu as pltpu
from jax.experimental.pallas import tpu_sc as plsc

NC = 2
NS = 16
NW = NC * NS
CH = 128

BE = 1000


def _sp(x):
    return jnp.maximum(x, 0.0) + jnp.log1p(jnp.exp(-jnp.abs(x)))


def _sc_gather_x(x, idx_i, idx_j):
    E = idx_i.shape[0]
    N, DIM = x.shape
    nchunks = E // CH
    per_tile = -(-nchunks // NW)
    mesh = plsc.VectorSubcoreMesh(core_axis_name="c", subcore_axis_name="s")

    @functools.partial(
        pl.kernel, mesh=mesh,
        out_type=(jax.ShapeDtypeStruct((E, DIM), x.dtype),
                  jax.ShapeDtypeStruct((E, DIM), x.dtype)),
        scratch_types=[pltpu.VMEM((CH,), jnp.int32),
                       pltpu.VMEM((CH, DIM), x.dtype)],
    )
    def k(x_hbm, ii_hbm, ij_hbm, gi_hbm, gj_hbm, idx_v, rows_v):
        wid = lax.axis_index("s") * NC + lax.axis_index("c")

        @pl.loop(0, per_tile)
        def _(t):
            kk = wid + NW * t

            @pl.when(kk < nchunks)
            def _():
                base = kk * CH
                pltpu.sync_copy(ii_hbm.at[pl.ds(base, CH)], idx_v)
                pltpu.sync_copy(x_hbm.at[idx_v], rows_v)
                pltpu.sync_copy(rows_v, gi_hbm.at[pl.ds(base, CH)])
                pltpu.sync_copy(ij_hbm.at[pl.ds(base, CH)], idx_v)
                pltpu.sync_copy(x_hbm.at[idx_v], rows_v)
                pltpu.sync_copy(rows_v, gj_hbm.at[pl.ds(base, CH)])

    return k(x, idx_i, idx_j)


def _sc_denom_scatter(ex16, idx_i, n_nodes):
    E = ex16.shape[0]
    nchunks = E // CH
    per_core = nchunks // NC
    per_tile = -(-per_core // NS)
    n_pad = -(-n_nodes // (8 * NS)) * 8 * NS
    rows_per_tile = n_pad // NS
    mesh = plsc.VectorSubcoreMesh(core_axis_name="c", subcore_axis_name="s")
    zrows = jnp.zeros((rows_per_tile, 16), jnp.float32)

    @functools.partial(
        pl.kernel, mesh=mesh,
        out_type=jax.ShapeDtypeStruct((NC, n_pad, 16), jnp.float32),
        scratch_types=[pltpu.VMEM((CH,), jnp.int32),
                       pltpu.VMEM((CH, 16), jnp.float32),
                       pltpu.VMEM_SHARED((n_pad, 16), jnp.float32)],
        compiler_params=pltpu.CompilerParams(use_tc_tiling_on_sc=False),
    )
    def k(ex_hbm, ii_hbm, z_hbm, den_hbm, idx_v, rows_v, acc_shared):
        c = lax.axis_index("c")
        s = lax.axis_index("s")
        pltpu.sync_copy(z_hbm, acc_shared.at[pl.ds(s * rows_per_tile, rows_per_tile)])
        plsc.subcore_barrier()

        @pl.loop(0, per_tile)
        def _(t):
            kk = s + NS * t

            @pl.when(kk < per_core)
            def _():
                base = (c * per_core + kk) * CH
                pltpu.sync_copy(ii_hbm.at[pl.ds(base, CH)], idx_v)
                pltpu.sync_copy(ex_hbm.at[pl.ds(base, CH)], rows_v)
                pltpu.sync_copy(rows_v, acc_shared.at[idx_v], add=True)

        plsc.subcore_barrier()
        sl = pl.ds(s * rows_per_tile, rows_per_tile)
        pltpu.sync_copy(acc_shared.at[sl], den_hbm.at[c].at[sl])

    return k(ex16, idx_i, zrows)


def _sc_denom_gather(den, idx_i):
    E = idx_i.shape[0]
    nchunks = E // CH
    per_tile = -(-nchunks // NW)
    mesh = plsc.VectorSubcoreMesh(core_axis_name="c", subcore_axis_name="s")

    @functools.partial(
        pl.kernel, mesh=mesh,
        out_type=jax.ShapeDtypeStruct((E, 16), jnp.float32),
        scratch_types=[pltpu.VMEM((CH,), jnp.int32),
                       pltpu.VMEM((CH, 16), jnp.float32)],
        compiler_params=pltpu.CompilerParams(use_tc_tiling_on_sc=False),
    )
    def k(den_hbm, ii_hbm, dg_hbm, idx_v, rows_v):
        wid = lax.axis_index("s") * NC + lax.axis_index("c")

        @pl.loop(0, per_tile)
        def _(t):
            kk = wid + NW * t

            @pl.when(kk < nchunks)
            def _():
                base = kk * CH
                pltpu.sync_copy(ii_hbm.at[pl.ds(base, CH)], idx_v)
                pltpu.sync_copy(den_hbm.at[idx_v], rows_v)
                pltpu.sync_copy(rows_v, dg_hbm.at[pl.ds(base, CH)])

    return k(den, idx_i)


def _sc_aggregate(msgr, idx_i, n_nodes):
    E, DIM = msgr.shape
    nchunks = E // CH
    per_core = nchunks // NC
    per_tile = -(-per_core // NS)
    n_pad = -(-n_nodes // (8 * NS)) * 8 * NS
    rows_per_tile = n_pad // NS
    mesh = plsc.VectorSubcoreMesh(core_axis_name="c", subcore_axis_name="s")
    zrows = jnp.zeros((rows_per_tile, DIM), jnp.float32)

    @functools.partial(
        pl.kernel, mesh=mesh,
        out_type=jax.ShapeDtypeStruct((NC, n_pad, DIM), jnp.float32),
        scratch_types=[pltpu.VMEM((CH,), jnp.int32),
                       pltpu.VMEM((CH, DIM), jnp.float32),
                       pltpu.VMEM_SHARED((n_pad, DIM), jnp.float32)],
    )
    def k(m_hbm, ii_hbm, z_hbm, agg_hbm, idx_v, rows_v, acc_shared):
        c = lax.axis_index("c")
        s = lax.axis_index("s")
        pltpu.sync_copy(z_hbm, acc_shared.at[pl.ds(s * rows_per_tile, rows_per_tile)])
        plsc.subcore_barrier()

        @pl.loop(0, per_tile)
        def _(t):
            kk = s + NS * t

            @pl.when(kk < per_core)
            def _():
                base = (c * per_core + kk) * CH
                pltpu.sync_copy(ii_hbm.at[pl.ds(base, CH)], idx_v)
                pltpu.sync_copy(m_hbm.at[pl.ds(base, CH)], rows_v)
                pltpu.sync_copy(rows_v, acc_shared.at[idx_v], add=True)

        plsc.subcore_barrier()
        sl = pl.ds(s * rows_per_tile, rows_per_tile)
        pltpu.sync_copy(acc_shared.at[sl], agg_hbm.at[c].at[sl])

    return k(msgr, idx_i, zrows)


def _tc_edge(gxi, gxj, ea, Wb, af, onesb):
    E, DIM = ea.shape
    HD = Wb.shape[1]
    H = onesb.shape[1]
    nb = E // BE

    def body(gxi_ref, gxj_ref, ea_ref, w_ref, af_ref, ones_ref,
             oj_ref, al_ref, s_ref, ss_ref):
        eab = ea_ref[...].astype(jnp.bfloat16)
        ci = jnp.concatenate([gxi_ref[...].astype(jnp.bfloat16), eab], axis=1)
        cj = jnp.concatenate([gxj_ref[...].astype(jnp.bfloat16), eab], axis=1)
        w = w_ref[...]
        ui = jnp.dot(ci, w, preferred_element_type=jnp.float32)
        uj = jnp.dot(cj, w, preferred_element_type=jnp.float32)
        oi = _sp(ui)
        oj = _sp(uj)
        oj_ref[...] = oj.astype(jnp.bfloat16)
        v = oi * af_ref[0:1, :] + oj * af_ref[1:2, :]
        al = jnp.dot(v.astype(jnp.bfloat16), ones_ref[...],
                     preferred_element_type=jnp.float32)
        al = _sp(al)
        al_ref[...] = al
        s_ref[...] = al.sum(axis=0).reshape(1, 1, H)
        ss_ref[...] = (al * al).sum(axis=0).reshape(1, 1, H)

    return pl.pallas_call(
        body,
        grid=(nb,),
        in_specs=[
            pl.BlockSpec((BE, DIM), lambda i: (i, 0)),
            pl.BlockSpec((BE, DIM), lambda i: (i, 0)),
            pl.BlockSpec((BE, DIM), lambda i: (i, 0)),
            pl.BlockSpec((2 * DIM, HD), lambda i: (0, 0)),
            pl.BlockSpec((8, HD), lambda i: (0, 0)),
            pl.BlockSpec((HD, H), lambda i: (0, 0)),
        ],
        out_specs=[
            pl.BlockSpec((BE, HD), lambda i: (i, 0)),
            pl.BlockSpec((BE, H), lambda i: (i, 0)),
            pl.BlockSpec((1, 1, H), lambda i: (i, 0, 0)),
            pl.BlockSpec((1, 1, H), lambda i: (i, 0, 0)),
        ],
        out_shape=[
            jax.ShapeDtypeStruct((E, HD), jnp.bfloat16),
            jax.ShapeDtypeStruct((E, H), jnp.float32),
            jax.ShapeDtypeStruct((nb, 1, H), jnp.float32),
            jax.ShapeDtypeStruct((nb, 1, H), jnp.float32),
        ],
    )(gxi, gxj, ea, Wb, af, onesb)


def _tc_bn_exp(alpha, params):
    E, H = alpha.shape
    nb = E // BE

    def body(al_ref, p_ref, ex_ref):
        al = al_ref[...]
        mean = p_ref[0:1, :]
        rstdg = p_ref[1:2, :]
        beta = p_ref[2:3, :]
        z = (al - mean) * rstdg + beta
        ex = jnp.exp(_sp(z))
        ex_ref[...] = jnp.concatenate(
            [ex, jnp.zeros((BE, 16 - H), jnp.float32)], axis=1)

    return pl.pallas_call(
        body,
        grid=(nb,),
        in_specs=[
            pl.BlockSpec((BE, H), lambda i: (i, 0)),
            pl.BlockSpec((8, H), lambda i: (0, 0)),
        ],
        out_specs=pl.BlockSpec((BE, 16), lambda i: (i, 0)),
        out_shape=jax.ShapeDtypeStruct((E, 16), jnp.float32),
    )(alpha, params)


def _tc_densum(den):
    NP = den.shape[1]
    BN = NP // 8

    def body(d_ref, o_ref):
        o_ref[...] = d_ref[0] + d_ref[1]

    return pl.pallas_call(
        body,
        grid=(8,),
        in_specs=[pl.BlockSpec((2, BN, 16), lambda i: (0, i, 0))],
        out_specs=pl.BlockSpec((BN, 16), lambda i: (i, 0)),
        out_shape=jax.ShapeDtypeStruct((NP, 16), jnp.float32),
    )(den)


def _tc_combine(oj, ex16, dg):
    E, HD = oj.shape
    H = 4
    DIM = HD // H
    nb = E // BE

    def body(oj_ref, ex_ref, d_ref, m_ref):
        w = ex_ref[:, :H] / (d_ref[:, :H] + 1e-16) * 0.25
        oj = oj_ref[...].astype(jnp.float32)
        acc = oj[:, 0:DIM] * w[:, 0:1]
        for h in range(1, H):
            acc = acc + oj[:, h * DIM:(h + 1) * DIM] * w[:, h:h + 1]
        m_ref[...] = acc

    return pl.pallas_call(
        body,
        grid=(nb,),
        in_specs=[
            pl.BlockSpec((BE, HD), lambda i: (i, 0)),
            pl.BlockSpec((BE, 16), lambda i: (i, 0)),
            pl.BlockSpec((BE, 16), lambda i: (i, 0)),
        ],
        out_specs=pl.BlockSpec((BE, DIM), lambda i: (i, 0)),
        out_shape=jax.ShapeDtypeStruct((E, DIM), jnp.float32),
    )(oj, ex16, dg)


def _tc_finalize(p0, p1, biasb):
    N, DIM = p0.shape
    BN = 1000
    nb = N // BN

    def body(a_ref, b_ref, bias_ref, o_ref):
        o_ref[...] = a_ref[...] + b_ref[...] + bias_ref[0:1, :]

    return pl.pallas_call(
        body,
        grid=(nb,),
        in_specs=[
            pl.BlockSpec((BN, DIM), lambda i: (i, 0)),
            pl.BlockSpec((BN, DIM), lambda i: (i, 0)),
            pl.BlockSpec((8, DIM), lambda i: (0, 0)),
        ],
        out_specs=pl.BlockSpec((BN, DIM), lambda i: (i, 0)),
        out_shape=jax.ShapeDtypeStruct((N, DIM), jnp.float32),
    )(p0, p1, biasb)


def kernel(x, edge_index, edge_attr, W, att, bias, bn_gamma, bn_beta):
    N, DIM = x.shape
    E = edge_attr.shape[0]
    H = att.shape[1]

    idx_i = edge_index[0]
    idx_j = edge_index[1]
    eye = jnp.eye(H, dtype=jnp.float32)
    af = jnp.zeros((8, H * DIM), jnp.float32)
    af = af.at[0].set(att[0, :, :DIM].reshape(-1)).at[1].set(att[0, :, DIM:].reshape(-1))
    onesb = (jnp.ones((H, DIM, 1)) * eye[:, None, :]).reshape(H * DIM, H).astype(jnp.bfloat16)
    Wb = W.astype(jnp.bfloat16)

    gxi, gxj = _sc_gather_x(x, idx_i, idx_j)
    oj, alpha, s_part, ss_part = _tc_edge(gxi, gxj, edge_attr, Wb, af, onesb)

    s = s_part.sum(axis=(0, 1))
    ss = ss_part.sum(axis=(0, 1))
    mean = s / E
    var = ss / E - mean * mean
    rstdg = bn_gamma / jnp.sqrt(var + 1e-5)
    params = jnp.zeros((8, H), jnp.float32)
    params = params.at[0].set(mean).at[1].set(rstdg).at[2].set(bn_beta)

    ex16 = _tc_bn_exp(alpha, params)
    den = _sc_denom_scatter(ex16, idx_i, N)
    dg = _sc_denom_gather(_tc_densum(den), idx_i)
    msgr = _tc_combine(oj, ex16, dg)
    agg = _sc_aggregate(msgr, idx_i, N)
    biasb = jnp.broadcast_to(bias, (8, DIM))
    return _tc_finalize(agg[0, :N], agg[1, :N], biasb)


# speedup vs baseline: 16.5036x; 1.1190x over previous
import functools

import jax
import jax.numpy as jnp
from jax import lax
from jax.experimental import pallas as pl
from jax.experimental.pallas import tpu as pltpu
from jax.experimental.pallas import tpu_sc as plsc

NC = 2
NS = 16
NW = NC * NS
CH = 128

BE = 1000


def _sp(x):
    return jnp.maximum(x, 0.0) + jnp.log1p(jnp.exp(-jnp.abs(x)))


def _sc_gather_x(x, idx_i, idx_j):
    E = idx_i.shape[0]
    N, DIM = x.shape
    nchunks = E // CH
    per_tile = -(-nchunks // NW)
    mesh = plsc.VectorSubcoreMesh(core_axis_name="c", subcore_axis_name="s")

    @functools.partial(
        pl.kernel, mesh=mesh,
        out_type=(jax.ShapeDtypeStruct((E, DIM), x.dtype),
                  jax.ShapeDtypeStruct((E, DIM), x.dtype)),
        scratch_types=[pltpu.VMEM((CH,), jnp.int32),
                       pltpu.VMEM((CH, DIM), x.dtype)],
    )
    def k(x_hbm, ii_hbm, ij_hbm, gi_hbm, gj_hbm, idx_v, rows_v):
        wid = lax.axis_index("s") * NC + lax.axis_index("c")

        @pl.loop(0, per_tile)
        def _(t):
            kk = wid + NW * t

            @pl.when(kk < nchunks)
            def _():
                base = kk * CH
                pltpu.sync_copy(ii_hbm.at[pl.ds(base, CH)], idx_v)
                pltpu.sync_copy(x_hbm.at[idx_v], rows_v)
                pltpu.sync_copy(rows_v, gi_hbm.at[pl.ds(base, CH)])
                pltpu.sync_copy(ij_hbm.at[pl.ds(base, CH)], idx_v)
                pltpu.sync_copy(x_hbm.at[idx_v], rows_v)
                pltpu.sync_copy(rows_v, gj_hbm.at[pl.ds(base, CH)])

    return k(x, idx_i, idx_j)


def _sc_denom_scatter(ex16, idx_i, n_nodes):
    E = ex16.shape[0]
    nchunks = E // CH
    per_core = nchunks // NC
    per_tile = -(-per_core // NS)
    n_pad = -(-n_nodes // (8 * NS)) * 8 * NS
    rows_per_tile = n_pad // NS
    mesh = plsc.VectorSubcoreMesh(core_axis_name="c", subcore_axis_name="s")
    zrows = jnp.zeros((rows_per_tile, 16), jnp.float32)

    @functools.partial(
        pl.kernel, mesh=mesh,
        out_type=jax.ShapeDtypeStruct((NC, n_pad, 16), jnp.float32),
        scratch_types=[pltpu.VMEM((CH,), jnp.int32),
                       pltpu.VMEM((CH, 16), jnp.float32),
                       pltpu.VMEM_SHARED((n_pad, 16), jnp.float32)],
        compiler_params=pltpu.CompilerParams(use_tc_tiling_on_sc=False),
    )
    def k(ex_hbm, ii_hbm, z_hbm, den_hbm, idx_v, rows_v, acc_shared):
        c = lax.axis_index("c")
        s = lax.axis_index("s")
        pltpu.sync_copy(z_hbm, acc_shared.at[pl.ds(s * rows_per_tile, rows_per_tile)])
        plsc.subcore_barrier()

        @pl.loop(0, per_tile)
        def _(t):
            kk = s + NS * t

            @pl.when(kk < per_core)
            def _():
                base = (c * per_core + kk) * CH
                pltpu.sync_copy(ii_hbm.at[pl.ds(base, CH)], idx_v)
                pltpu.sync_copy(ex_hbm.at[pl.ds(base, CH)], rows_v)
                pltpu.sync_copy(rows_v, acc_shared.at[idx_v], add=True)

        plsc.subcore_barrier()
        sl = pl.ds(s * rows_per_tile, rows_per_tile)
        pltpu.sync_copy(acc_shared.at[sl], den_hbm.at[c].at[sl])

    return k(ex16, idx_i, zrows)


def _sc_denom_gather(den, idx_i):
    E = idx_i.shape[0]
    nchunks = E // CH
    per_tile = -(-nchunks // NW)
    mesh = plsc.VectorSubcoreMesh(core_axis_name="c", subcore_axis_name="s")

    @functools.partial(
        pl.kernel, mesh=mesh,
        out_type=jax.ShapeDtypeStruct((E, 16), jnp.float32),
        scratch_types=[pltpu.VMEM((CH,), jnp.int32),
                       pltpu.VMEM((CH, 16), jnp.float32)],
        compiler_params=pltpu.CompilerParams(use_tc_tiling_on_sc=False),
    )
    def k(den_hbm, ii_hbm, dg_hbm, idx_v, rows_v):
        wid = lax.axis_index("s") * NC + lax.axis_index("c")

        @pl.loop(0, per_tile)
        def _(t):
            kk = wid + NW * t

            @pl.when(kk < nchunks)
            def _():
                base = kk * CH
                pltpu.sync_copy(ii_hbm.at[pl.ds(base, CH)], idx_v)
                pltpu.sync_copy(den_hbm.at[idx_v], rows_v)
                pltpu.sync_copy(rows_v, dg_hbm.at[pl.ds(base, CH)])

    return k(den, idx_i)


def _sc_aggregate(msgr, idx_i, n_nodes):
    E, DIM = msgr.shape
    nchunks = E // CH
    per_core = nchunks // NC
    per_tile = -(-per_core // NS)
    n_pad = -(-n_nodes // (8 * NS)) * 8 * NS
    rows_per_tile = n_pad // NS
    mesh = plsc.VectorSubcoreMesh(core_axis_name="c", subcore_axis_name="s")
    zrows = jnp.zeros((rows_per_tile, DIM), jnp.float32)

    @functools.partial(
        pl.kernel, mesh=mesh,
        out_type=jax.ShapeDtypeStruct((NC, n_pad, DIM), jnp.float32),
        scratch_types=[pltpu.VMEM((CH,), jnp.int32),
                       pltpu.VMEM((CH, DIM), jnp.float32),
                       pltpu.VMEM_SHARED((n_pad, DIM), jnp.float32)],
    )
    def k(m_hbm, ii_hbm, z_hbm, agg_hbm, idx_v, rows_v, acc_shared):
        c = lax.axis_index("c")
        s = lax.axis_index("s")
        pltpu.sync_copy(z_hbm, acc_shared.at[pl.ds(s * rows_per_tile, rows_per_tile)])
        plsc.subcore_barrier()

        @pl.loop(0, per_tile)
        def _(t):
            kk = s + NS * t

            @pl.when(kk < per_core)
            def _():
                base = (c * per_core + kk) * CH
                pltpu.sync_copy(ii_hbm.at[pl.ds(base, CH)], idx_v)
                pltpu.sync_copy(m_hbm.at[pl.ds(base, CH)], rows_v)
                pltpu.sync_copy(rows_v, acc_shared.at[idx_v], add=True)

        plsc.subcore_barrier()
        sl = pl.ds(s * rows_per_tile, rows_per_tile)
        pltpu.sync_copy(acc_shared.at[sl], agg_hbm.at[c].at[sl])

    return k(msgr, idx_i, zrows)


def _tc_edge(gxi, gxj, ea, Wb, af, onesb):
    E, DIM = ea.shape
    HD = Wb.shape[1]
    H = onesb.shape[1]
    nb = E // BE

    def body(gxi_ref, gxj_ref, ea_ref, w_ref, af_ref, ones_ref,
             oj_ref, al_ref, s_ref, ss_ref):
        eab = ea_ref[...].astype(jnp.bfloat16)
        ci = jnp.concatenate([gxi_ref[...].astype(jnp.bfloat16), eab], axis=1)
        cj = jnp.concatenate([gxj_ref[...].astype(jnp.bfloat16), eab], axis=1)
        w = w_ref[...]
        ui = jnp.dot(ci, w, preferred_element_type=jnp.float32).astype(jnp.bfloat16)
        uj = jnp.dot(cj, w, preferred_element_type=jnp.float32).astype(jnp.bfloat16)
        oi = _sp(ui)
        oj = _sp(uj)
        oj_ref[...] = oj
        v = oi * af_ref[0:1, :] + oj * af_ref[1:2, :]
        al = jnp.dot(v, ones_ref[...], preferred_element_type=jnp.float32)
        al = _sp(al)
        al_ref[...] = al
        s_ref[...] = al.sum(axis=0).reshape(1, 1, H)
        ss_ref[...] = (al * al).sum(axis=0).reshape(1, 1, H)

    return pl.pallas_call(
        body,
        grid=(nb,),
        in_specs=[
            pl.BlockSpec((BE, DIM), lambda i: (i, 0)),
            pl.BlockSpec((BE, DIM), lambda i: (i, 0)),
            pl.BlockSpec((BE, DIM), lambda i: (i, 0)),
            pl.BlockSpec((2 * DIM, HD), lambda i: (0, 0)),
            pl.BlockSpec((8, HD), lambda i: (0, 0)),
            pl.BlockSpec((HD, H), lambda i: (0, 0)),
        ],
        out_specs=[
            pl.BlockSpec((BE, HD), lambda i: (i, 0)),
            pl.BlockSpec((BE, H), lambda i: (i, 0)),
            pl.BlockSpec((1, 1, H), lambda i: (i, 0, 0)),
            pl.BlockSpec((1, 1, H), lambda i: (i, 0, 0)),
        ],
        out_shape=[
            jax.ShapeDtypeStruct((E, HD), jnp.bfloat16),
            jax.ShapeDtypeStruct((E, H), jnp.float32),
            jax.ShapeDtypeStruct((nb, 1, H), jnp.float32),
            jax.ShapeDtypeStruct((nb, 1, H), jnp.float32),
        ],
    )(gxi, gxj, ea, Wb, af, onesb)


def _tc_bn_exp(alpha, params):
    E, H = alpha.shape
    nb = E // BE

    def body(al_ref, p_ref, ex_ref):
        al = al_ref[...]
        mean = p_ref[0:1, :]
        rstdg = p_ref[1:2, :]
        beta = p_ref[2:3, :]
        z = (al - mean) * rstdg + beta
        ex = jnp.exp(_sp(z))
        ex_ref[...] = jnp.concatenate(
            [ex, jnp.zeros((BE, 16 - H), jnp.float32)], axis=1)

    return pl.pallas_call(
        body,
        grid=(nb,),
        in_specs=[
            pl.BlockSpec((BE, H), lambda i: (i, 0)),
            pl.BlockSpec((8, H), lambda i: (0, 0)),
        ],
        out_specs=pl.BlockSpec((BE, 16), lambda i: (i, 0)),
        out_shape=jax.ShapeDtypeStruct((E, 16), jnp.float32),
    )(alpha, params)


def _tc_densum(den):
    NP = den.shape[1]
    BN = NP // 8

    def body(d_ref, o_ref):
        o_ref[...] = d_ref[0] + d_ref[1]

    return pl.pallas_call(
        body,
        grid=(8,),
        in_specs=[pl.BlockSpec((2, BN, 16), lambda i: (0, i, 0))],
        out_specs=pl.BlockSpec((BN, 16), lambda i: (i, 0)),
        out_shape=jax.ShapeDtypeStruct((NP, 16), jnp.float32),
    )(den)


def _tc_combine(oj, ex16, dg):
    E, HD = oj.shape
    H = 4
    DIM = HD // H
    nb = E // BE

    def body(oj_ref, ex_ref, d_ref, m_ref):
        w = ex_ref[:, :H] / (d_ref[:, :H] + 1e-16) * 0.25
        oj = oj_ref[...].astype(jnp.float32)
        acc = oj[:, 0:DIM] * w[:, 0:1]
        for h in range(1, H):
            acc = acc + oj[:, h * DIM:(h + 1) * DIM] * w[:, h:h + 1]
        m_ref[...] = acc

    return pl.pallas_call(
        body,
        grid=(nb,),
        in_specs=[
            pl.BlockSpec((BE, HD), lambda i: (i, 0)),
            pl.BlockSpec((BE, 16), lambda i: (i, 0)),
            pl.BlockSpec((BE, 16), lambda i: (i, 0)),
        ],
        out_specs=pl.BlockSpec((BE, DIM), lambda i: (i, 0)),
        out_shape=jax.ShapeDtypeStruct((E, DIM), jnp.float32),
    )(oj, ex16, dg)


def _tc_finalize(p0, p1, biasb):
    N, DIM = p0.shape
    BN = 1000
    nb = N // BN

    def body(a_ref, b_ref, bias_ref, o_ref):
        o_ref[...] = a_ref[...] + b_ref[...] + bias_ref[0:1, :]

    return pl.pallas_call(
        body,
        grid=(nb,),
        in_specs=[
            pl.BlockSpec((BN, DIM), lambda i: (i, 0)),
            pl.BlockSpec((BN, DIM), lambda i: (i, 0)),
            pl.BlockSpec((8, DIM), lambda i: (0, 0)),
        ],
        out_specs=pl.BlockSpec((BN, DIM), lambda i: (i, 0)),
        out_shape=jax.ShapeDtypeStruct((N, DIM), jnp.float32),
    )(p0, p1, biasb)


def kernel(x, edge_index, edge_attr, W, att, bias, bn_gamma, bn_beta):
    N, DIM = x.shape
    E = edge_attr.shape[0]
    H = att.shape[1]

    idx_i = edge_index[0]
    idx_j = edge_index[1]
    eye = jnp.eye(H, dtype=jnp.float32)
    af = jnp.zeros((8, H * DIM), jnp.float32)
    af = af.at[0].set(att[0, :, :DIM].reshape(-1)).at[1].set(att[0, :, DIM:].reshape(-1))
    af = af.astype(jnp.bfloat16)
    onesb = (jnp.ones((H, DIM, 1)) * eye[:, None, :]).reshape(H * DIM, H).astype(jnp.bfloat16)
    Wb = W.astype(jnp.bfloat16)

    gxi, gxj = _sc_gather_x(x, idx_i, idx_j)
    oj, alpha, s_part, ss_part = _tc_edge(gxi, gxj, edge_attr, Wb, af, onesb)

    s = s_part.sum(axis=(0, 1))
    ss = ss_part.sum(axis=(0, 1))
    mean = s / E
    var = ss / E - mean * mean
    rstdg = bn_gamma / jnp.sqrt(var + 1e-5)
    params = jnp.zeros((8, H), jnp.float32)
    params = params.at[0].set(mean).at[1].set(rstdg).at[2].set(bn_beta)

    ex16 = _tc_bn_exp(alpha, params)
    den = _sc_denom_scatter(ex16, idx_i, N)
    dg = _sc_denom_gather(_tc_densum(den), idx_i)
    msgr = _tc_combine(oj, ex16, dg)
    agg = _sc_aggregate(msgr, idx_i, N)
    biasb = jnp.broadcast_to(bias, (8, DIM))
    return _tc_finalize(agg[0, :N], agg[1, :N], biasb)


# speedup vs baseline: 16.7142x; 1.0128x over previous
import functools

import jax
import jax.numpy as jnp
from jax import lax
from jax.experimental import pallas as pl
from jax.experimental.pallas import tpu as pltpu
from jax.experimental.pallas import tpu_sc as plsc

NC = 2
NS = 16
NW = NC * NS
CH = 128

BE = 1000


def _sp(x):
    return jnp.maximum(x, 0.0) + jnp.log1p(jnp.exp(-jnp.abs(x)))


def _sc_gather_x(x, idx_i, idx_j):
    E = idx_i.shape[0]
    N, DIM = x.shape
    nchunks = E // CH
    per_tile = -(-nchunks // NW)
    mesh = plsc.VectorSubcoreMesh(core_axis_name="c", subcore_axis_name="s")

    @functools.partial(
        pl.kernel, mesh=mesh,
        out_type=(jax.ShapeDtypeStruct((E, DIM), x.dtype),
                  jax.ShapeDtypeStruct((E, DIM), x.dtype)),
        scratch_types=[pltpu.VMEM((CH,), jnp.int32),
                       pltpu.VMEM((CH, DIM), x.dtype)],
    )
    def k(x_hbm, ii_hbm, ij_hbm, gi_hbm, gj_hbm, idx_v, rows_v):
        wid = lax.axis_index("s") * NC + lax.axis_index("c")

        @pl.loop(0, per_tile)
        def _(t):
            kk = wid + NW * t

            @pl.when(kk < nchunks)
            def _():
                base = kk * CH
                pltpu.sync_copy(ii_hbm.at[pl.ds(base, CH)], idx_v)
                pltpu.sync_copy(x_hbm.at[idx_v], rows_v)
                pltpu.sync_copy(rows_v, gi_hbm.at[pl.ds(base, CH)])
                pltpu.sync_copy(ij_hbm.at[pl.ds(base, CH)], idx_v)
                pltpu.sync_copy(x_hbm.at[idx_v], rows_v)
                pltpu.sync_copy(rows_v, gj_hbm.at[pl.ds(base, CH)])

    return k(x, idx_i, idx_j)


def _sc_denom_scatter(ex16, idx_i, n_nodes):
    E = ex16.shape[0]
    nchunks = E // CH
    per_core = nchunks // NC
    per_tile = -(-per_core // NS)
    n_pad = -(-n_nodes // (8 * NS)) * 8 * NS
    rows_per_tile = n_pad // NS
    mesh = plsc.VectorSubcoreMesh(core_axis_name="c", subcore_axis_name="s")
    zrows = jnp.zeros((rows_per_tile, 16), jnp.float32)

    @functools.partial(
        pl.kernel, mesh=mesh,
        out_type=jax.ShapeDtypeStruct((NC, n_pad, 16), jnp.float32),
        scratch_types=[pltpu.VMEM((CH,), jnp.int32),
                       pltpu.VMEM((CH, 16), jnp.float32),
                       pltpu.VMEM_SHARED((n_pad, 16), jnp.float32)],
        compiler_params=pltpu.CompilerParams(use_tc_tiling_on_sc=False),
    )
    def k(ex_hbm, ii_hbm, z_hbm, den_hbm, idx_v, rows_v, acc_shared):
        c = lax.axis_index("c")
        s = lax.axis_index("s")
        pltpu.sync_copy(z_hbm, acc_shared.at[pl.ds(s * rows_per_tile, rows_per_tile)])
        plsc.subcore_barrier()

        @pl.loop(0, per_tile)
        def _(t):
            kk = s + NS * t

            @pl.when(kk < per_core)
            def _():
                base = (c * per_core + kk) * CH
                pltpu.sync_copy(ii_hbm.at[pl.ds(base, CH)], idx_v)
                pltpu.sync_copy(ex_hbm.at[pl.ds(base, CH)], rows_v)
                pltpu.sync_copy(rows_v, acc_shared.at[idx_v], add=True)

        plsc.subcore_barrier()
        sl = pl.ds(s * rows_per_tile, rows_per_tile)
        pltpu.sync_copy(acc_shared.at[sl], den_hbm.at[c].at[sl])

    return k(ex16, idx_i, zrows)


def _sc_denom_gather(den, idx_i):
    E = idx_i.shape[0]
    nchunks = E // CH
    per_tile = -(-nchunks // NW)
    mesh = plsc.VectorSubcoreMesh(core_axis_name="c", subcore_axis_name="s")

    @functools.partial(
        pl.kernel, mesh=mesh,
        out_type=jax.ShapeDtypeStruct((E, 16), jnp.float32),
        scratch_types=[pltpu.VMEM((CH,), jnp.int32),
                       pltpu.VMEM((CH, 16), jnp.float32)],
        compiler_params=pltpu.CompilerParams(use_tc_tiling_on_sc=False),
    )
    def k(den_hbm, ii_hbm, dg_hbm, idx_v, rows_v):
        wid = lax.axis_index("s") * NC + lax.axis_index("c")

        @pl.loop(0, per_tile)
        def _(t):
            kk = wid + NW * t

            @pl.when(kk < nchunks)
            def _():
                base = kk * CH
                pltpu.sync_copy(ii_hbm.at[pl.ds(base, CH)], idx_v)
                pltpu.sync_copy(den_hbm.at[idx_v], rows_v)
                pltpu.sync_copy(rows_v, dg_hbm.at[pl.ds(base, CH)])

    return k(den, idx_i)


def _sc_aggregate(msgr, idx_i, n_nodes):
    E, DIM = msgr.shape
    nchunks = E // CH
    per_core = nchunks // NC
    per_tile = -(-per_core // NS)
    n_pad = -(-n_nodes // (8 * NS)) * 8 * NS
    rows_per_tile = n_pad // NS
    mesh = plsc.VectorSubcoreMesh(core_axis_name="c", subcore_axis_name="s")
    zrows = jnp.zeros((rows_per_tile, DIM), jnp.float32)

    @functools.partial(
        pl.kernel, mesh=mesh,
        out_type=jax.ShapeDtypeStruct((NC, n_pad, DIM), jnp.float32),
        scratch_types=[pltpu.VMEM((CH,), jnp.int32),
                       pltpu.VMEM((CH, DIM), jnp.float32),
                       pltpu.VMEM_SHARED((n_pad, DIM), jnp.float32)],
    )
    def k(m_hbm, ii_hbm, z_hbm, agg_hbm, idx_v, rows_v, acc_shared):
        c = lax.axis_index("c")
        s = lax.axis_index("s")
        pltpu.sync_copy(z_hbm, acc_shared.at[pl.ds(s * rows_per_tile, rows_per_tile)])
        plsc.subcore_barrier()

        @pl.loop(0, per_tile)
        def _(t):
            kk = s + NS * t

            @pl.when(kk < per_core)
            def _():
                base = (c * per_core + kk) * CH
                pltpu.sync_copy(ii_hbm.at[pl.ds(base, CH)], idx_v)
                pltpu.sync_copy(m_hbm.at[pl.ds(base, CH)], rows_v)
                pltpu.sync_copy(rows_v, acc_shared.at[idx_v], add=True)

        plsc.subcore_barrier()
        sl = pl.ds(s * rows_per_tile, rows_per_tile)
        pltpu.sync_copy(acc_shared.at[sl], agg_hbm.at[c].at[sl])

    return k(msgr, idx_i, zrows)


def _tc_edge(gxi, gxj, ea, Wb, af, onesb):
    E, DIM = ea.shape
    HD = Wb.shape[1]
    H = onesb.shape[1]
    nb = E // BE

    def body(gxi_ref, gxj_ref, ea_ref, w_ref, af_ref, ones_ref,
             oj_ref, al_ref, s_ref, ss_ref):
        eab = ea_ref[...].astype(jnp.bfloat16)
        ci = jnp.concatenate([gxi_ref[...].astype(jnp.bfloat16), eab], axis=1)
        cj = jnp.concatenate([gxj_ref[...].astype(jnp.bfloat16), eab], axis=1)
        w = w_ref[...]
        ui = jnp.dot(ci, w, preferred_element_type=jnp.float32).astype(jnp.bfloat16)
        uj = jnp.dot(cj, w, preferred_element_type=jnp.float32).astype(jnp.bfloat16)
        oi = _sp(ui)
        oj = _sp(uj)
        oj_ref[...] = oj
        v = oi * af_ref[0:1, :] + oj * af_ref[1:2, :]
        al = jnp.dot(v, ones_ref[...], preferred_element_type=jnp.float32)
        al = _sp(al)
        al_ref[...] = al
        s_ref[...] = al.sum(axis=0).reshape(1, 1, H)
        ss_ref[...] = (al * al).sum(axis=0).reshape(1, 1, H)

    return pl.pallas_call(
        body,
        grid=(nb,),
        in_specs=[
            pl.BlockSpec((BE, DIM), lambda i: (i, 0)),
            pl.BlockSpec((BE, DIM), lambda i: (i, 0)),
            pl.BlockSpec((BE, DIM), lambda i: (i, 0)),
            pl.BlockSpec((2 * DIM, HD), lambda i: (0, 0)),
            pl.BlockSpec((8, HD), lambda i: (0, 0)),
            pl.BlockSpec((HD, H), lambda i: (0, 0)),
        ],
        out_specs=[
            pl.BlockSpec((BE, HD), lambda i: (i, 0)),
            pl.BlockSpec((BE, H), lambda i: (i, 0)),
            pl.BlockSpec((1, 1, H), lambda i: (i, 0, 0)),
            pl.BlockSpec((1, 1, H), lambda i: (i, 0, 0)),
        ],
        out_shape=[
            jax.ShapeDtypeStruct((E, HD), jnp.bfloat16),
            jax.ShapeDtypeStruct((E, H), jnp.float32),
            jax.ShapeDtypeStruct((nb, 1, H), jnp.float32),
            jax.ShapeDtypeStruct((nb, 1, H), jnp.float32),
        ],
    )(gxi, gxj, ea, Wb, af, onesb)


def _tc_bn_exp(alpha, params):
    E, H = alpha.shape
    nb = E // BE

    def body(al_ref, p_ref, ex_ref):
        al = al_ref[...]
        mean = p_ref[0:1, :]
        rstdg = p_ref[1:2, :]
        beta = p_ref[2:3, :]
        z = (al - mean) * rstdg + beta
        ex = jnp.exp(_sp(z))
        ex_ref[...] = jnp.concatenate(
            [ex, jnp.zeros((BE, 16 - H), jnp.float32)], axis=1)

    return pl.pallas_call(
        body,
        grid=(nb,),
        in_specs=[
            pl.BlockSpec((BE, H), lambda i: (i, 0)),
            pl.BlockSpec((8, H), lambda i: (0, 0)),
        ],
        out_specs=pl.BlockSpec((BE, 16), lambda i: (i, 0)),
        out_shape=jax.ShapeDtypeStruct((E, 16), jnp.float32),
    )(alpha, params)


def _tc_densum(den):
    NP = den.shape[1]
    BN = NP // 8

    def body(d_ref, o_ref):
        o_ref[...] = d_ref[0] + d_ref[1]

    return pl.pallas_call(
        body,
        grid=(8,),
        in_specs=[pl.BlockSpec((2, BN, 16), lambda i: (0, i, 0))],
        out_specs=pl.BlockSpec((BN, 16), lambda i: (i, 0)),
        out_shape=jax.ShapeDtypeStruct((NP, 16), jnp.float32),
    )(den)


def _tc_combine(oj, ex16, dg):
    E, HD = oj.shape
    H = 4
    DIM = HD // H
    nb = E // BE

    def body(oj_ref, ex_ref, d_ref, m_ref):
        w = (ex_ref[:, :H] / (d_ref[:, :H] + 1e-16) * 0.25).astype(jnp.bfloat16)
        oj = oj_ref[...]
        acc = oj[:, 0:DIM] * w[:, 0:1]
        for h in range(1, H):
            acc = acc + oj[:, h * DIM:(h + 1) * DIM] * w[:, h:h + 1]
        m_ref[...] = acc.astype(jnp.float32)

    return pl.pallas_call(
        body,
        grid=(nb,),
        in_specs=[
            pl.BlockSpec((BE, HD), lambda i: (i, 0)),
            pl.BlockSpec((BE, 16), lambda i: (i, 0)),
            pl.BlockSpec((BE, 16), lambda i: (i, 0)),
        ],
        out_specs=pl.BlockSpec((BE, DIM), lambda i: (i, 0)),
        out_shape=jax.ShapeDtypeStruct((E, DIM), jnp.float32),
    )(oj, ex16, dg)


def _tc_finalize(p0, p1, biasb):
    N, DIM = p0.shape
    BN = 1000
    nb = N // BN

    def body(a_ref, b_ref, bias_ref, o_ref):
        o_ref[...] = a_ref[...] + b_ref[...] + bias_ref[0:1, :]

    return pl.pallas_call(
        body,
        grid=(nb,),
        in_specs=[
            pl.BlockSpec((BN, DIM), lambda i: (i, 0)),
            pl.BlockSpec((BN, DIM), lambda i: (i, 0)),
            pl.BlockSpec((8, DIM), lambda i: (0, 0)),
        ],
        out_specs=pl.BlockSpec((BN, DIM), lambda i: (i, 0)),
        out_shape=jax.ShapeDtypeStruct((N, DIM), jnp.float32),
    )(p0, p1, biasb)


def kernel(x, edge_index, edge_attr, W, att, bias, bn_gamma, bn_beta):
    N, DIM = x.shape
    E = edge_attr.shape[0]
    H = att.shape[1]

    idx_i = edge_index[0]
    idx_j = edge_index[1]
    eye = jnp.eye(H, dtype=jnp.float32)
    af = jnp.zeros((8, H * DIM), jnp.float32)
    af = af.at[0].set(att[0, :, :DIM].reshape(-1)).at[1].set(att[0, :, DIM:].reshape(-1))
    af = af.astype(jnp.bfloat16)
    onesb = (jnp.ones((H, DIM, 1)) * eye[:, None, :]).reshape(H * DIM, H).astype(jnp.bfloat16)
    Wb = W.astype(jnp.bfloat16)

    gxi, gxj = _sc_gather_x(x, idx_i, idx_j)
    oj, alpha, s_part, ss_part = _tc_edge(gxi, gxj, edge_attr, Wb, af, onesb)

    s = s_part.sum(axis=(0, 1))
    ss = ss_part.sum(axis=(0, 1))
    mean = s / E
    var = ss / E - mean * mean
    rstdg = bn_gamma / jnp.sqrt(var + 1e-5)
    params = jnp.zeros((8, H), jnp.float32)
    params = params.at[0].set(mean).at[1].set(rstdg).at[2].set(bn_beta)

    ex16 = _tc_bn_exp(alpha, params)
    den = _sc_denom_scatter(ex16, idx_i, N)
    dg = _sc_denom_gather(_tc_densum(den), idx_i)
    msgr = _tc_combine(oj, ex16, dg)
    agg = _sc_aggregate(msgr, idx_i, N)
    biasb = jnp.broadcast_to(bias, (8, DIM))
    return _tc_finalize(agg[0, :N], agg[1, :N], biasb)


# speedup vs baseline: 18.1389x; 1.0852x over previous
import functools

import jax
import jax.numpy as jnp
from jax import lax
from jax.experimental import pallas as pl
from jax.experimental.pallas import tpu as pltpu
from jax.experimental.pallas import tpu_sc as plsc

NC = 2
NS = 16
NW = NC * NS
CH = 128

BE = 1000


def _sp(x):
    return jnp.maximum(x, 0.0) + jnp.log1p(jnp.exp(-jnp.abs(x)))


def _sc_gather_x(x, idx_i, idx_j):
    E = idx_i.shape[0]
    N, DIM = x.shape
    ept = E // NW
    nfull = ept // CH
    tail = ept - nfull * CH
    mesh = plsc.VectorSubcoreMesh(core_axis_name="c", subcore_axis_name="s")

    @functools.partial(
        pl.kernel, mesh=mesh,
        out_type=(jax.ShapeDtypeStruct((E, DIM), x.dtype),
                  jax.ShapeDtypeStruct((E, DIM), x.dtype)),
        scratch_types=[pltpu.VMEM((ept,), jnp.int32),
                       pltpu.VMEM((ept,), jnp.int32),
                       pltpu.VMEM((CH, DIM), x.dtype),
                       pltpu.VMEM((CH, DIM), x.dtype),
                       pltpu.SemaphoreType.DMA,
                       pltpu.SemaphoreType.DMA,
                       pltpu.SemaphoreType.DMA,
                       pltpu.SemaphoreType.DMA],
    )
    def k(x_hbm, ii_hbm, ij_hbm, gi_hbm, gj_hbm,
          idxi_v, idxj_v, buf0, buf1, g0, g1, w0, w1):
        wid = lax.axis_index("s") * NC + lax.axis_index("c")
        base = wid * ept
        pltpu.sync_copy(ii_hbm.at[pl.ds(base, ept)], idxi_v)
        pltpu.sync_copy(ij_hbm.at[pl.ds(base, ept)], idxj_v)
        bufs = (buf0, buf1)
        gsem = (g0, g1)
        wsem = (w0, w1)

        for idx_v, out_hbm in ((idxi_v, gi_hbm), (idxj_v, gj_hbm)):
            def gat(cur, b):
                return pltpu.make_async_copy(
                    x_hbm.at[idx_v.at[pl.ds(cur * CH, CH)]], bufs[b], gsem[b])

            def wrb(cur, b):
                return pltpu.make_async_copy(
                    bufs[b], out_hbm.at[pl.ds(base + cur * CH, CH)], wsem[b])

            for b in range(2):
                gat(b, b).start()

            @pl.loop(0, nfull, step=2)
            def _(t):
                for b in range(2):
                    cur = t + b
                    gat(cur, b).wait()
                    wrb(cur, b).start()
                    wrb(cur, b).wait()

                    @pl.when(cur + 2 < nfull)
                    def _():
                        gat(cur + 2, b).start()

            if tail:
                pltpu.sync_copy(x_hbm.at[idx_v.at[pl.ds(nfull * CH, tail)]],
                                buf0.at[pl.ds(0, tail)])
                pltpu.sync_copy(buf0.at[pl.ds(0, tail)],
                                out_hbm.at[pl.ds(base + nfull * CH, tail)])

    return k(x, idx_i, idx_j)


def _sc_denom_scatter(ex16, idx_i, n_nodes):
    E = ex16.shape[0]
    nchunks = E // CH
    per_core = nchunks // NC
    per_tile = -(-per_core // NS)
    n_pad = -(-n_nodes // (8 * NS)) * 8 * NS
    rows_per_tile = n_pad // NS
    mesh = plsc.VectorSubcoreMesh(core_axis_name="c", subcore_axis_name="s")
    zrows = jnp.zeros((rows_per_tile, 16), jnp.float32)

    @functools.partial(
        pl.kernel, mesh=mesh,
        out_type=jax.ShapeDtypeStruct((NC, n_pad, 16), jnp.float32),
        scratch_types=[pltpu.VMEM((CH,), jnp.int32),
                       pltpu.VMEM((CH, 16), jnp.float32),
                       pltpu.VMEM_SHARED((n_pad, 16), jnp.float32)],
        compiler_params=pltpu.CompilerParams(use_tc_tiling_on_sc=False),
    )
    def k(ex_hbm, ii_hbm, z_hbm, den_hbm, idx_v, rows_v, acc_shared):
        c = lax.axis_index("c")
        s = lax.axis_index("s")
        pltpu.sync_copy(z_hbm, acc_shared.at[pl.ds(s * rows_per_tile, rows_per_tile)])
        plsc.subcore_barrier()

        @pl.loop(0, per_tile)
        def _(t):
            kk = s + NS * t

            @pl.when(kk < per_core)
            def _():
                base = (c * per_core + kk) * CH
                pltpu.sync_copy(ii_hbm.at[pl.ds(base, CH)], idx_v)
                pltpu.sync_copy(ex_hbm.at[pl.ds(base, CH)], rows_v)
                pltpu.sync_copy(rows_v, acc_shared.at[idx_v], add=True)

        plsc.subcore_barrier()
        sl = pl.ds(s * rows_per_tile, rows_per_tile)
        pltpu.sync_copy(acc_shared.at[sl], den_hbm.at[c].at[sl])

    return k(ex16, idx_i, zrows)


def _sc_denom_gather(den, idx_i):
    E = idx_i.shape[0]
    nchunks = E // CH
    per_tile = -(-nchunks // NW)
    mesh = plsc.VectorSubcoreMesh(core_axis_name="c", subcore_axis_name="s")

    @functools.partial(
        pl.kernel, mesh=mesh,
        out_type=jax.ShapeDtypeStruct((E, 16), jnp.float32),
        scratch_types=[pltpu.VMEM((CH,), jnp.int32),
                       pltpu.VMEM((CH, 16), jnp.float32)],
        compiler_params=pltpu.CompilerParams(use_tc_tiling_on_sc=False),
    )
    def k(den_hbm, ii_hbm, dg_hbm, idx_v, rows_v):
        wid = lax.axis_index("s") * NC + lax.axis_index("c")

        @pl.loop(0, per_tile)
        def _(t):
            kk = wid + NW * t

            @pl.when(kk < nchunks)
            def _():
                base = kk * CH
                pltpu.sync_copy(ii_hbm.at[pl.ds(base, CH)], idx_v)
                pltpu.sync_copy(den_hbm.at[idx_v], rows_v)
                pltpu.sync_copy(rows_v, dg_hbm.at[pl.ds(base, CH)])

    return k(den, idx_i)


def _sc_aggregate(msgr, idx_i, n_nodes):
    E, DIM = msgr.shape
    nchunks = E // CH
    per_core = nchunks // NC
    per_tile = -(-per_core // NS)
    n_pad = -(-n_nodes // (8 * NS)) * 8 * NS
    rows_per_tile = n_pad // NS
    mesh = plsc.VectorSubcoreMesh(core_axis_name="c", subcore_axis_name="s")
    zrows = jnp.zeros((rows_per_tile, DIM), jnp.float32)

    @functools.partial(
        pl.kernel, mesh=mesh,
        out_type=jax.ShapeDtypeStruct((NC, n_pad, DIM), jnp.float32),
        scratch_types=[pltpu.VMEM((CH,), jnp.int32),
                       pltpu.VMEM((CH, DIM), jnp.float32),
                       pltpu.VMEM_SHARED((n_pad, DIM), jnp.float32)],
    )
    def k(m_hbm, ii_hbm, z_hbm, agg_hbm, idx_v, rows_v, acc_shared):
        c = lax.axis_index("c")
        s = lax.axis_index("s")
        pltpu.sync_copy(z_hbm, acc_shared.at[pl.ds(s * rows_per_tile, rows_per_tile)])
        plsc.subcore_barrier()

        @pl.loop(0, per_tile)
        def _(t):
            kk = s + NS * t

            @pl.when(kk < per_core)
            def _():
                base = (c * per_core + kk) * CH
                pltpu.sync_copy(ii_hbm.at[pl.ds(base, CH)], idx_v)
                pltpu.sync_copy(m_hbm.at[pl.ds(base, CH)], rows_v)
                pltpu.sync_copy(rows_v, acc_shared.at[idx_v], add=True)

        plsc.subcore_barrier()
        sl = pl.ds(s * rows_per_tile, rows_per_tile)
        pltpu.sync_copy(acc_shared.at[sl], agg_hbm.at[c].at[sl])

    return k(msgr, idx_i, zrows)


def _tc_edge(gxi, gxj, ea, Wb, af, onesb):
    E, DIM = ea.shape
    HD = Wb.shape[1]
    H = onesb.shape[1]
    nb = E // BE

    def body(gxi_ref, gxj_ref, ea_ref, w_ref, af_ref, ones_ref,
             oj_ref, al_ref, s_ref, ss_ref):
        eab = ea_ref[...].astype(jnp.bfloat16)
        ci = jnp.concatenate([gxi_ref[...].astype(jnp.bfloat16), eab], axis=1)
        cj = jnp.concatenate([gxj_ref[...].astype(jnp.bfloat16), eab], axis=1)
        w = w_ref[...]
        ui = jnp.dot(ci, w, preferred_element_type=jnp.float32).astype(jnp.bfloat16)
        uj = jnp.dot(cj, w, preferred_element_type=jnp.float32).astype(jnp.bfloat16)
        oi = _sp(ui)
        oj = _sp(uj)
        oj_ref[...] = oj
        v = oi * af_ref[0:1, :] + oj * af_ref[1:2, :]
        al = jnp.dot(v, ones_ref[...], preferred_element_type=jnp.float32)
        al = _sp(al)
        al_ref[...] = al
        s_ref[...] = al.sum(axis=0).reshape(1, 1, H)
        ss_ref[...] = (al * al).sum(axis=0).reshape(1, 1, H)

    return pl.pallas_call(
        body,
        grid=(nb,),
        in_specs=[
            pl.BlockSpec((BE, DIM), lambda i: (i, 0)),
            pl.BlockSpec((BE, DIM), lambda i: (i, 0)),
            pl.BlockSpec((BE, DIM), lambda i: (i, 0)),
            pl.BlockSpec((2 * DIM, HD), lambda i: (0, 0)),
            pl.BlockSpec((8, HD), lambda i: (0, 0)),
            pl.BlockSpec((HD, H), lambda i: (0, 0)),
        ],
        out_specs=[
            pl.BlockSpec((BE, HD), lambda i: (i, 0)),
            pl.BlockSpec((BE, H), lambda i: (i, 0)),
            pl.BlockSpec((1, 1, H), lambda i: (i, 0, 0)),
            pl.BlockSpec((1, 1, H), lambda i: (i, 0, 0)),
        ],
        out_shape=[
            jax.ShapeDtypeStruct((E, HD), jnp.bfloat16),
            jax.ShapeDtypeStruct((E, H), jnp.float32),
            jax.ShapeDtypeStruct((nb, 1, H), jnp.float32),
            jax.ShapeDtypeStruct((nb, 1, H), jnp.float32),
        ],
    )(gxi, gxj, ea, Wb, af, onesb)


def _tc_bn_exp(alpha, params):
    E, H = alpha.shape
    nb = E // BE

    def body(al_ref, p_ref, ex_ref):
        al = al_ref[...]
        mean = p_ref[0:1, :]
        rstdg = p_ref[1:2, :]
        beta = p_ref[2:3, :]
        z = (al - mean) * rstdg + beta
        ex = jnp.exp(_sp(z))
        ex_ref[...] = jnp.concatenate(
            [ex, jnp.zeros((BE, 16 - H), jnp.float32)], axis=1)

    return pl.pallas_call(
        body,
        grid=(nb,),
        in_specs=[
            pl.BlockSpec((BE, H), lambda i: (i, 0)),
            pl.BlockSpec((8, H), lambda i: (0, 0)),
        ],
        out_specs=pl.BlockSpec((BE, 16), lambda i: (i, 0)),
        out_shape=jax.ShapeDtypeStruct((E, 16), jnp.float32),
    )(alpha, params)


def _tc_densum(den):
    NP = den.shape[1]
    BN = NP // 8

    def body(d_ref, o_ref):
        o_ref[...] = d_ref[0] + d_ref[1]

    return pl.pallas_call(
        body,
        grid=(8,),
        in_specs=[pl.BlockSpec((2, BN, 16), lambda i: (0, i, 0))],
        out_specs=pl.BlockSpec((BN, 16), lambda i: (i, 0)),
        out_shape=jax.ShapeDtypeStruct((NP, 16), jnp.float32),
    )(den)


def _tc_combine(oj, ex16, dg):
    E, HD = oj.shape
    H = 4
    DIM = HD // H
    nb = E // BE

    def body(oj_ref, ex_ref, d_ref, m_ref):
        w = (ex_ref[:, :H] / (d_ref[:, :H] + 1e-16) * 0.25).astype(jnp.bfloat16)
        oj = oj_ref[...]
        acc = oj[:, 0:DIM] * w[:, 0:1]
        for h in range(1, H):
            acc = acc + oj[:, h * DIM:(h + 1) * DIM] * w[:, h:h + 1]
        m_ref[...] = acc.astype(jnp.float32)

    return pl.pallas_call(
        body,
        grid=(nb,),
        in_specs=[
            pl.BlockSpec((BE, HD), lambda i: (i, 0)),
            pl.BlockSpec((BE, 16), lambda i: (i, 0)),
            pl.BlockSpec((BE, 16), lambda i: (i, 0)),
        ],
        out_specs=pl.BlockSpec((BE, DIM), lambda i: (i, 0)),
        out_shape=jax.ShapeDtypeStruct((E, DIM), jnp.float32),
    )(oj, ex16, dg)


def _tc_finalize(p0, p1, biasb):
    N, DIM = p0.shape
    BN = 1000
    nb = N // BN

    def body(a_ref, b_ref, bias_ref, o_ref):
        o_ref[...] = a_ref[...] + b_ref[...] + bias_ref[0:1, :]

    return pl.pallas_call(
        body,
        grid=(nb,),
        in_specs=[
            pl.BlockSpec((BN, DIM), lambda i: (i, 0)),
            pl.BlockSpec((BN, DIM), lambda i: (i, 0)),
            pl.BlockSpec((8, DIM), lambda i: (0, 0)),
        ],
        out_specs=pl.BlockSpec((BN, DIM), lambda i: (i, 0)),
        out_shape=jax.ShapeDtypeStruct((N, DIM), jnp.float32),
    )(p0, p1, biasb)


def kernel(x, edge_index, edge_attr, W, att, bias, bn_gamma, bn_beta):
    N, DIM = x.shape
    E = edge_attr.shape[0]
    H = att.shape[1]

    idx_i = edge_index[0]
    idx_j = edge_index[1]
    eye = jnp.eye(H, dtype=jnp.float32)
    af = jnp.zeros((8, H * DIM), jnp.float32)
    af = af.at[0].set(att[0, :, :DIM].reshape(-1)).at[1].set(att[0, :, DIM:].reshape(-1))
    af = af.astype(jnp.bfloat16)
    onesb = (jnp.ones((H, DIM, 1)) * eye[:, None, :]).reshape(H * DIM, H).astype(jnp.bfloat16)
    Wb = W.astype(jnp.bfloat16)

    gxi, gxj = _sc_gather_x(x, idx_i, idx_j)
    oj, alpha, s_part, ss_part = _tc_edge(gxi, gxj, edge_attr, Wb, af, onesb)

    s = s_part.sum(axis=(0, 1))
    ss = ss_part.sum(axis=(0, 1))
    mean = s / E
    var = ss / E - mean * mean
    rstdg = bn_gamma / jnp.sqrt(var + 1e-5)
    params = jnp.zeros((8, H), jnp.float32)
    params = params.at[0].set(mean).at[1].set(rstdg).at[2].set(bn_beta)

    ex16 = _tc_bn_exp(alpha, params)
    den = _sc_denom_scatter(ex16, idx_i, N)
    dg = _sc_denom_gather(_tc_densum(den), idx_i)
    msgr = _tc_combine(oj, ex16, dg)
    agg = _sc_aggregate(msgr, idx_i, N)
    biasb = jnp.broadcast_to(bias, (8, DIM))
    return _tc_finalize(agg[0, :N], agg[1, :N], biasb)


# speedup vs baseline: 20.3010x; 1.1192x over previous
import functools

import jax
import jax.numpy as jnp
from jax import lax
from jax.experimental import pallas as pl
from jax.experimental.pallas import tpu as pltpu
from jax.experimental.pallas import tpu_sc as plsc

NC = 2
NS = 16
NW = NC * NS
CH = 128

BE = 1000


def _sp(x):
    return jnp.maximum(x, 0.0) + jnp.log1p(jnp.exp(-jnp.abs(x)))


def _sc_gather_x(x, idx_i, idx_j):
    E = idx_i.shape[0]
    N, DIM = x.shape
    ept = E // NW
    nfull = ept // CH
    tail = ept - nfull * CH
    mesh = plsc.VectorSubcoreMesh(core_axis_name="c", subcore_axis_name="s")

    @functools.partial(
        pl.kernel, mesh=mesh,
        out_type=(jax.ShapeDtypeStruct((E, DIM), x.dtype),
                  jax.ShapeDtypeStruct((E, DIM), x.dtype)),
        scratch_types=[pltpu.VMEM((ept,), jnp.int32),
                       pltpu.VMEM((ept,), jnp.int32),
                       pltpu.VMEM((CH, DIM), x.dtype),
                       pltpu.VMEM((CH, DIM), x.dtype),
                       pltpu.SemaphoreType.DMA,
                       pltpu.SemaphoreType.DMA,
                       pltpu.SemaphoreType.DMA,
                       pltpu.SemaphoreType.DMA],
    )
    def k(x_hbm, ii_hbm, ij_hbm, gi_hbm, gj_hbm,
          idxi_v, idxj_v, buf0, buf1, g0, g1, w0, w1):
        wid = lax.axis_index("s") * NC + lax.axis_index("c")
        base = wid * ept
        pltpu.sync_copy(ii_hbm.at[pl.ds(base, ept)], idxi_v)
        pltpu.sync_copy(ij_hbm.at[pl.ds(base, ept)], idxj_v)
        bufs = (buf0, buf1)
        gsem = (g0, g1)
        wsem = (w0, w1)

        for idx_v, out_hbm in ((idxi_v, gi_hbm), (idxj_v, gj_hbm)):
            def gat(cur, b):
                return pltpu.make_async_copy(
                    x_hbm.at[idx_v.at[pl.ds(cur * CH, CH)]], bufs[b], gsem[b])

            def wrb(cur, b):
                return pltpu.make_async_copy(
                    bufs[b], out_hbm.at[pl.ds(base + cur * CH, CH)], wsem[b])

            for b in range(2):
                gat(b, b).start()

            @pl.loop(0, nfull, step=2)
            def _(t):
                for b in range(2):
                    cur = t + b
                    gat(cur, b).wait()
                    wrb(cur, b).start()
                    wrb(cur, b).wait()

                    @pl.when(cur + 2 < nfull)
                    def _():
                        gat(cur + 2, b).start()

            if tail:
                pltpu.sync_copy(x_hbm.at[idx_v.at[pl.ds(nfull * CH, tail)]],
                                buf0.at[pl.ds(0, tail)])
                pltpu.sync_copy(buf0.at[pl.ds(0, tail)],
                                out_hbm.at[pl.ds(base + nfull * CH, tail)])

    return k(x, idx_i, idx_j)


def _sc_scatter_add(vals, idx_i, n_nodes, width):
    E = vals.shape[0]
    nchunks = E // CH
    per_core = nchunks // NC
    per_tile = -(-per_core // NS)
    n_pad = -(-n_nodes // (8 * NS)) * 8 * NS
    rows_per_tile = n_pad // NS
    mesh = plsc.VectorSubcoreMesh(core_axis_name="c", subcore_axis_name="s")
    zrows = jnp.zeros((rows_per_tile, width), jnp.float32)

    @functools.partial(
        pl.kernel, mesh=mesh,
        out_type=jax.ShapeDtypeStruct((NC, n_pad, width), jnp.float32),
        scratch_types=[pltpu.VMEM((CH,), jnp.int32),
                       pltpu.VMEM((CH,), jnp.int32),
                       pltpu.VMEM((CH, width), jnp.float32),
                       pltpu.VMEM((CH, width), jnp.float32),
                       pltpu.VMEM_SHARED((n_pad, width), jnp.float32),
                       pltpu.SemaphoreType.DMA,
                       pltpu.SemaphoreType.DMA,
                       pltpu.SemaphoreType.DMA,
                       pltpu.SemaphoreType.DMA,
                       pltpu.SemaphoreType.DMA,
                       pltpu.SemaphoreType.DMA],
        compiler_params=(pltpu.CompilerParams(use_tc_tiling_on_sc=False)
                         if width < 128 else None),
    )
    def k(v_hbm, ii_hbm, z_hbm, out_hbm, idx0, idx1, buf0, buf1, acc_shared,
          i0, i1, g0, g1, w0, w1):
        c = lax.axis_index("c")
        s = lax.axis_index("s")
        pltpu.sync_copy(z_hbm, acc_shared.at[pl.ds(s * rows_per_tile, rows_per_tile)])
        plsc.subcore_barrier()
        idxs = (idx0, idx1)
        bufs = (buf0, buf1)
        isem = (i0, i1)
        gsem = (g0, g1)
        wsem = (w0, w1)

        def chunk_of(t):
            return c * per_core + s + NS * t

        def load(t, b):
            base = chunk_of(t) * CH
            pltpu.make_async_copy(ii_hbm.at[pl.ds(base, CH)], idxs[b],
                                  isem[b]).start()
            pltpu.make_async_copy(v_hbm.at[pl.ds(base, CH)], bufs[b],
                                  gsem[b]).start()

        def load_wait(t, b):
            base = chunk_of(t) * CH
            pltpu.make_async_copy(ii_hbm.at[pl.ds(base, CH)], idxs[b],
                                  isem[b]).wait()
            pltpu.make_async_copy(v_hbm.at[pl.ds(base, CH)], bufs[b],
                                  gsem[b]).wait()

        for b in range(2):
            @pl.when(s + NS * b < per_core)
            def _():
                load(b, b)

        @pl.loop(0, per_tile, step=2)
        def _(t):
            for b in range(2):
                tt = t + b
                kk = s + NS * tt

                @pl.when(kk < per_core)
                def _():
                    load_wait(tt, b)
                    pltpu.async_copy(bufs[b], acc_shared.at[idxs[b]],
                                     wsem[b], add=True)
                    pltpu.make_async_copy(bufs[b], acc_shared.at[idxs[b]],
                                          wsem[b]).wait()

                    @pl.when(s + NS * (tt + 2) < per_core)
                    def _():
                        load(tt + 2, b)

        plsc.subcore_barrier()
        sl = pl.ds(s * rows_per_tile, rows_per_tile)
        pltpu.sync_copy(acc_shared.at[sl], out_hbm.at[c].at[sl])

    return k(vals, idx_i, zrows)


def _sc_denom_gather(den, idx_i):
    E = idx_i.shape[0]
    ept = E // NW
    nfull = ept // CH
    tail = ept - nfull * CH
    mesh = plsc.VectorSubcoreMesh(core_axis_name="c", subcore_axis_name="s")

    @functools.partial(
        pl.kernel, mesh=mesh,
        out_type=jax.ShapeDtypeStruct((E, 16), jnp.float32),
        scratch_types=[pltpu.VMEM((ept,), jnp.int32),
                       pltpu.VMEM((CH, 16), jnp.float32),
                       pltpu.VMEM((CH, 16), jnp.float32),
                       pltpu.SemaphoreType.DMA,
                       pltpu.SemaphoreType.DMA,
                       pltpu.SemaphoreType.DMA,
                       pltpu.SemaphoreType.DMA],
        compiler_params=pltpu.CompilerParams(use_tc_tiling_on_sc=False),
    )
    def k(den_hbm, ii_hbm, dg_hbm, idx_v, buf0, buf1, g0, g1, w0, w1):
        wid = lax.axis_index("s") * NC + lax.axis_index("c")
        base = wid * ept
        pltpu.sync_copy(ii_hbm.at[pl.ds(base, ept)], idx_v)
        bufs = (buf0, buf1)
        gsem = (g0, g1)
        wsem = (w0, w1)

        def gat(cur, b):
            return pltpu.make_async_copy(
                den_hbm.at[idx_v.at[pl.ds(cur * CH, CH)]], bufs[b], gsem[b])

        def wrb(cur, b):
            return pltpu.make_async_copy(
                bufs[b], dg_hbm.at[pl.ds(base + cur * CH, CH)], wsem[b])

        for b in range(2):
            gat(b, b).start()

        @pl.loop(0, nfull, step=2)
        def _(t):
            for b in range(2):
                cur = t + b
                gat(cur, b).wait()
                wrb(cur, b).start()
                wrb(cur, b).wait()

                @pl.when(cur + 2 < nfull)
                def _():
                    gat(cur + 2, b).start()

        if tail:
            pltpu.sync_copy(den_hbm.at[idx_v.at[pl.ds(nfull * CH, tail)]],
                            buf0.at[pl.ds(0, tail)])
            pltpu.sync_copy(buf0.at[pl.ds(0, tail)],
                            dg_hbm.at[pl.ds(base + nfull * CH, tail)])

    return k(den, idx_i)


def _tc_edge(gxi, gxj, ea, Wb, af, onesb):
    E, DIM = ea.shape
    HD = Wb.shape[1]
    H = onesb.shape[1]
    nb = E // BE

    def body(gxi_ref, gxj_ref, ea_ref, w_ref, af_ref, ones_ref,
             oj_ref, al_ref, s_ref, ss_ref):
        eab = ea_ref[...].astype(jnp.bfloat16)
        ci = jnp.concatenate([gxi_ref[...].astype(jnp.bfloat16), eab], axis=1)
        cj = jnp.concatenate([gxj_ref[...].astype(jnp.bfloat16), eab], axis=1)
        w = w_ref[...]
        ui = jnp.dot(ci, w, preferred_element_type=jnp.float32).astype(jnp.bfloat16)
        uj = jnp.dot(cj, w, preferred_element_type=jnp.float32).astype(jnp.bfloat16)
        oi = _sp(ui)
        oj = _sp(uj)
        oj_ref[...] = oj
        v = oi * af_ref[0:1, :] + oj * af_ref[1:2, :]
        al = jnp.dot(v, ones_ref[...], preferred_element_type=jnp.float32)
        al = _sp(al)
        al_ref[...] = al
        s_ref[...] = al.sum(axis=0).reshape(1, 1, H)
        ss_ref[...] = (al * al).sum(axis=0).reshape(1, 1, H)

    return pl.pallas_call(
        body,
        grid=(nb,),
        in_specs=[
            pl.BlockSpec((BE, DIM), lambda i: (i, 0)),
            pl.BlockSpec((BE, DIM), lambda i: (i, 0)),
            pl.BlockSpec((BE, DIM), lambda i: (i, 0)),
            pl.BlockSpec((2 * DIM, HD), lambda i: (0, 0)),
            pl.BlockSpec((8, HD), lambda i: (0, 0)),
            pl.BlockSpec((HD, H), lambda i: (0, 0)),
        ],
        out_specs=[
            pl.BlockSpec((BE, HD), lambda i: (i, 0)),
            pl.BlockSpec((BE, H), lambda i: (i, 0)),
            pl.BlockSpec((1, 1, H), lambda i: (i, 0, 0)),
            pl.BlockSpec((1, 1, H), lambda i: (i, 0, 0)),
        ],
        out_shape=[
            jax.ShapeDtypeStruct((E, HD), jnp.bfloat16),
            jax.ShapeDtypeStruct((E, H), jnp.float32),
            jax.ShapeDtypeStruct((nb, 1, H), jnp.float32),
            jax.ShapeDtypeStruct((nb, 1, H), jnp.float32),
        ],
    )(gxi, gxj, ea, Wb, af, onesb)


def _tc_bn_exp(alpha, params):
    E, H = alpha.shape
    nb = E // BE

    def body(al_ref, p_ref, ex_ref):
        al = al_ref[...]
        mean = p_ref[0:1, :]
        rstdg = p_ref[1:2, :]
        beta = p_ref[2:3, :]
        z = (al - mean) * rstdg + beta
        ex = jnp.exp(_sp(z))
        ex_ref[...] = jnp.concatenate(
            [ex, jnp.zeros((BE, 16 - H), jnp.float32)], axis=1)

    return pl.pallas_call(
        body,
        grid=(nb,),
        in_specs=[
            pl.BlockSpec((BE, H), lambda i: (i, 0)),
            pl.BlockSpec((8, H), lambda i: (0, 0)),
        ],
        out_specs=pl.BlockSpec((BE, 16), lambda i: (i, 0)),
        out_shape=jax.ShapeDtypeStruct((E, 16), jnp.float32),
    )(alpha, params)


def _tc_densum(den):
    NP = den.shape[1]
    BN = NP // 8

    def body(d_ref, o_ref):
        o_ref[...] = d_ref[0] + d_ref[1]

    return pl.pallas_call(
        body,
        grid=(8,),
        in_specs=[pl.BlockSpec((2, BN, 16), lambda i: (0, i, 0))],
        out_specs=pl.BlockSpec((BN, 16), lambda i: (i, 0)),
        out_shape=jax.ShapeDtypeStruct((NP, 16), jnp.float32),
    )(den)


def _tc_combine(oj, ex16, dg):
    E, HD = oj.shape
    H = 4
    DIM = HD // H
    nb = E // BE

    def body(oj_ref, ex_ref, d_ref, m_ref):
        w = (ex_ref[:, :H] / (d_ref[:, :H] + 1e-16) * 0.25).astype(jnp.bfloat16)
        oj = oj_ref[...]
        acc = oj[:, 0:DIM] * w[:, 0:1]
        for h in range(1, H):
            acc = acc + oj[:, h * DIM:(h + 1) * DIM] * w[:, h:h + 1]
        m_ref[...] = acc.astype(jnp.float32)

    return pl.pallas_call(
        body,
        grid=(nb,),
        in_specs=[
            pl.BlockSpec((BE, HD), lambda i: (i, 0)),
            pl.BlockSpec((BE, 16), lambda i: (i, 0)),
            pl.BlockSpec((BE, 16), lambda i: (i, 0)),
        ],
        out_specs=pl.BlockSpec((BE, DIM), lambda i: (i, 0)),
        out_shape=jax.ShapeDtypeStruct((E, DIM), jnp.float32),
    )(oj, ex16, dg)


def _tc_finalize(p0, p1, biasb):
    N, DIM = p0.shape
    BN = 1000
    nb = N // BN

    def body(a_ref, b_ref, bias_ref, o_ref):
        o_ref[...] = a_ref[...] + b_ref[...] + bias_ref[0:1, :]

    return pl.pallas_call(
        body,
        grid=(nb,),
        in_specs=[
            pl.BlockSpec((BN, DIM), lambda i: (i, 0)),
            pl.BlockSpec((BN, DIM), lambda i: (i, 0)),
            pl.BlockSpec((8, DIM), lambda i: (0, 0)),
        ],
        out_specs=pl.BlockSpec((BN, DIM), lambda i: (i, 0)),
        out_shape=jax.ShapeDtypeStruct((N, DIM), jnp.float32),
    )(p0, p1, biasb)


def kernel(x, edge_index, edge_attr, W, att, bias, bn_gamma, bn_beta):
    N, DIM = x.shape
    E = edge_attr.shape[0]
    H = att.shape[1]

    idx_i = edge_index[0]
    idx_j = edge_index[1]
    eye = jnp.eye(H, dtype=jnp.float32)
    af = jnp.zeros((8, H * DIM), jnp.float32)
    af = af.at[0].set(att[0, :, :DIM].reshape(-1)).at[1].set(att[0, :, DIM:].reshape(-1))
    af = af.astype(jnp.bfloat16)
    onesb = (jnp.ones((H, DIM, 1)) * eye[:, None, :]).reshape(H * DIM, H).astype(jnp.bfloat16)
    Wb = W.astype(jnp.bfloat16)

    gxi, gxj = _sc_gather_x(x, idx_i, idx_j)
    oj, alpha, s_part, ss_part = _tc_edge(gxi, gxj, edge_attr, Wb, af, onesb)

    s = s_part.sum(axis=(0, 1))
    ss = ss_part.sum(axis=(0, 1))
    mean = s / E
    var = ss / E - mean * mean
    rstdg = bn_gamma / jnp.sqrt(var + 1e-5)
    params = jnp.zeros((8, H), jnp.float32)
    params = params.at[0].set(mean).at[1].set(rstdg).at[2].set(bn_beta)

    ex16 = _tc_bn_exp(alpha, params)
    den = _sc_scatter_add(ex16, idx_i, N, 16)
    dg = _sc_denom_gather(_tc_densum(den), idx_i)
    msgr = _tc_combine(oj, ex16, dg)
    agg = _sc_scatter_add(msgr, idx_i, N, 128)
    biasb = jnp.broadcast_to(bias, (8, DIM))
    return _tc_finalize(agg[0, :N], agg[1, :N], biasb)


# speedup vs baseline: 23.6462x; 1.1648x over previous
import functools

import jax
import jax.numpy as jnp
from jax import lax
from jax.experimental import pallas as pl
from jax.experimental.pallas import tpu as pltpu
from jax.experimental.pallas import tpu_sc as plsc

NC = 2
NS = 16
NW = NC * NS
CH = 128

BE = 2000


def _sp(x):
    return jnp.maximum(x, 0.0) + jnp.log1p(jnp.exp(-jnp.abs(x)))


def _sc_gather_x(x, idx_i, idx_j):
    E = idx_i.shape[0]
    N, DIM = x.shape
    ept = E // NW
    nfull = ept // CH
    tail = ept - nfull * CH
    mesh = plsc.VectorSubcoreMesh(core_axis_name="c", subcore_axis_name="s")

    @functools.partial(
        pl.kernel, mesh=mesh,
        out_type=(jax.ShapeDtypeStruct((E, DIM), x.dtype),
                  jax.ShapeDtypeStruct((E, DIM), x.dtype)),
        scratch_types=[pltpu.VMEM((ept,), jnp.int32),
                       pltpu.VMEM((ept,), jnp.int32),
                       pltpu.VMEM((CH, DIM), x.dtype),
                       pltpu.VMEM((CH, DIM), x.dtype),
                       pltpu.SemaphoreType.DMA,
                       pltpu.SemaphoreType.DMA,
                       pltpu.SemaphoreType.DMA,
                       pltpu.SemaphoreType.DMA],
    )
    def k(x_hbm, ii_hbm, ij_hbm, gi_hbm, gj_hbm,
          idxi_v, idxj_v, buf0, buf1, g0, g1, w0, w1):
        wid = lax.axis_index("s") * NC + lax.axis_index("c")
        base = wid * ept
        pltpu.sync_copy(ii_hbm.at[pl.ds(base, ept)], idxi_v)
        pltpu.sync_copy(ij_hbm.at[pl.ds(base, ept)], idxj_v)
        bufs = (buf0, buf1)
        gsem = (g0, g1)
        wsem = (w0, w1)

        for idx_v, out_hbm in ((idxi_v, gi_hbm), (idxj_v, gj_hbm)):
            def gat(cur, b):
                return pltpu.make_async_copy(
                    x_hbm.at[idx_v.at[pl.ds(cur * CH, CH)]], bufs[b], gsem[b])

            def wrb(cur, b):
                return pltpu.make_async_copy(
                    bufs[b], out_hbm.at[pl.ds(base + cur * CH, CH)], wsem[b])

            for b in range(2):
                gat(b, b).start()

            @pl.loop(0, nfull, step=2)
            def _(t):
                for b in range(2):
                    cur = t + b
                    gat(cur, b).wait()
                    wrb(cur, b).start()
                    wrb(cur, b).wait()

                    @pl.when(cur + 2 < nfull)
                    def _():
                        gat(cur + 2, b).start()

            if tail:
                pltpu.sync_copy(x_hbm.at[idx_v.at[pl.ds(nfull * CH, tail)]],
                                buf0.at[pl.ds(0, tail)])
                pltpu.sync_copy(buf0.at[pl.ds(0, tail)],
                                out_hbm.at[pl.ds(base + nfull * CH, tail)])

    return k(x, idx_i, idx_j)


def _sc_scatter_add(vals, idx_i, n_nodes, width):
    E = vals.shape[0]
    nchunks = E // CH
    per_core = nchunks // NC
    per_tile = -(-per_core // NS)
    n_pad = -(-n_nodes // (8 * NS)) * 8 * NS
    rows_per_tile = n_pad // NS
    mesh = plsc.VectorSubcoreMesh(core_axis_name="c", subcore_axis_name="s")
    zrows = jnp.zeros((rows_per_tile, width), jnp.float32)

    @functools.partial(
        pl.kernel, mesh=mesh,
        out_type=jax.ShapeDtypeStruct((NC, n_pad, width), jnp.float32),
        scratch_types=[pltpu.VMEM((CH,), jnp.int32),
                       pltpu.VMEM((CH,), jnp.int32),
                       pltpu.VMEM((CH, width), jnp.float32),
                       pltpu.VMEM((CH, width), jnp.float32),
                       pltpu.VMEM_SHARED((n_pad, width), jnp.float32),
                       pltpu.SemaphoreType.DMA,
                       pltpu.SemaphoreType.DMA,
                       pltpu.SemaphoreType.DMA,
                       pltpu.SemaphoreType.DMA,
                       pltpu.SemaphoreType.DMA,
                       pltpu.SemaphoreType.DMA],
        compiler_params=(pltpu.CompilerParams(use_tc_tiling_on_sc=False)
                         if width < 128 else None),
    )
    def k(v_hbm, ii_hbm, z_hbm, out_hbm, idx0, idx1, buf0, buf1, acc_shared,
          i0, i1, g0, g1, w0, w1):
        c = lax.axis_index("c")
        s = lax.axis_index("s")
        pltpu.sync_copy(z_hbm, acc_shared.at[pl.ds(s * rows_per_tile, rows_per_tile)])
        plsc.subcore_barrier()
        idxs = (idx0, idx1)
        bufs = (buf0, buf1)
        isem = (i0, i1)
        gsem = (g0, g1)
        wsem = (w0, w1)

        def chunk_of(t):
            return c * per_core + s + NS * t

        def load(t, b):
            base = chunk_of(t) * CH
            pltpu.make_async_copy(ii_hbm.at[pl.ds(base, CH)], idxs[b],
                                  isem[b]).start()
            pltpu.make_async_copy(v_hbm.at[pl.ds(base, CH)], bufs[b],
                                  gsem[b]).start()

        def load_wait(t, b):
            base = chunk_of(t) * CH
            pltpu.make_async_copy(ii_hbm.at[pl.ds(base, CH)], idxs[b],
                                  isem[b]).wait()
            pltpu.make_async_copy(v_hbm.at[pl.ds(base, CH)], bufs[b],
                                  gsem[b]).wait()

        for b in range(2):
            @pl.when(s + NS * b < per_core)
            def _():
                load(b, b)

        @pl.loop(0, per_tile, step=2)
        def _(t):
            for b in range(2):
                tt = t + b
                kk = s + NS * tt

                @pl.when(kk < per_core)
                def _():
                    load_wait(tt, b)
                    pltpu.async_copy(bufs[b], acc_shared.at[idxs[b]],
                                     wsem[b], add=True)
                    pltpu.make_async_copy(bufs[b], acc_shared.at[idxs[b]],
                                          wsem[b]).wait()

                    @pl.when(s + NS * (tt + 2) < per_core)
                    def _():
                        load(tt + 2, b)

        plsc.subcore_barrier()
        sl = pl.ds(s * rows_per_tile, rows_per_tile)
        pltpu.sync_copy(acc_shared.at[sl], out_hbm.at[c].at[sl])

    return k(vals, idx_i, zrows)


def _sc_denom_gather(den, idx_i):
    E = idx_i.shape[0]
    ept = E // NW
    nfull = ept // CH
    tail = ept - nfull * CH
    mesh = plsc.VectorSubcoreMesh(core_axis_name="c", subcore_axis_name="s")

    @functools.partial(
        pl.kernel, mesh=mesh,
        out_type=jax.ShapeDtypeStruct((E, 16), jnp.float32),
        scratch_types=[pltpu.VMEM((ept,), jnp.int32),
                       pltpu.VMEM((CH, 16), jnp.float32),
                       pltpu.VMEM((CH, 16), jnp.float32),
                       pltpu.SemaphoreType.DMA,
                       pltpu.SemaphoreType.DMA,
                       pltpu.SemaphoreType.DMA,
                       pltpu.SemaphoreType.DMA],
        compiler_params=pltpu.CompilerParams(use_tc_tiling_on_sc=False),
    )
    def k(den_hbm, ii_hbm, dg_hbm, idx_v, buf0, buf1, g0, g1, w0, w1):
        wid = lax.axis_index("s") * NC + lax.axis_index("c")
        base = wid * ept
        pltpu.sync_copy(ii_hbm.at[pl.ds(base, ept)], idx_v)
        bufs = (buf0, buf1)
        gsem = (g0, g1)
        wsem = (w0, w1)

        def gat(cur, b):
            return pltpu.make_async_copy(
                den_hbm.at[idx_v.at[pl.ds(cur * CH, CH)]], bufs[b], gsem[b])

        def wrb(cur, b):
            return pltpu.make_async_copy(
                bufs[b], dg_hbm.at[pl.ds(base + cur * CH, CH)], wsem[b])

        for b in range(2):
            gat(b, b).start()

        @pl.loop(0, nfull, step=2)
        def _(t):
            for b in range(2):
                cur = t + b
                gat(cur, b).wait()
                wrb(cur, b).start()
                wrb(cur, b).wait()

                @pl.when(cur + 2 < nfull)
                def _():
                    gat(cur + 2, b).start()

        if tail:
            pltpu.sync_copy(den_hbm.at[idx_v.at[pl.ds(nfull * CH, tail)]],
                            buf0.at[pl.ds(0, tail)])
            pltpu.sync_copy(buf0.at[pl.ds(0, tail)],
                            dg_hbm.at[pl.ds(base + nfull * CH, tail)])

    return k(den, idx_i)


def _tc_edge(gxi, gxj, ea, Wb, af, onesb):
    E, DIM = ea.shape
    HD = Wb.shape[1]
    H = onesb.shape[1]
    nb = E // BE

    def body(gxi_ref, gxj_ref, ea_ref, w_ref, af_ref, ones_ref,
             oj_ref, al_ref, s_ref, ss_ref):
        eab = ea_ref[...].astype(jnp.bfloat16)
        ci = jnp.concatenate([gxi_ref[...].astype(jnp.bfloat16), eab], axis=1)
        cj = jnp.concatenate([gxj_ref[...].astype(jnp.bfloat16), eab], axis=1)
        w = w_ref[...]
        ui = jnp.dot(ci, w, preferred_element_type=jnp.float32).astype(jnp.bfloat16)
        uj = jnp.dot(cj, w, preferred_element_type=jnp.float32).astype(jnp.bfloat16)
        oi = _sp(ui)
        oj = _sp(uj)
        oj_ref[...] = oj
        v = oi * af_ref[0:1, :] + oj * af_ref[1:2, :]
        al = jnp.dot(v, ones_ref[...], preferred_element_type=jnp.float32)
        al = _sp(al)
        al_ref[...] = al
        s_ref[...] = al.sum(axis=0).reshape(1, 1, H)
        ss_ref[...] = (al * al).sum(axis=0).reshape(1, 1, H)

    return pl.pallas_call(
        body,
        grid=(nb,),
        in_specs=[
            pl.BlockSpec((BE, DIM), lambda i: (i, 0)),
            pl.BlockSpec((BE, DIM), lambda i: (i, 0)),
            pl.BlockSpec((BE, DIM), lambda i: (i, 0)),
            pl.BlockSpec((2 * DIM, HD), lambda i: (0, 0)),
            pl.BlockSpec((8, HD), lambda i: (0, 0)),
            pl.BlockSpec((HD, H), lambda i: (0, 0)),
        ],
        out_specs=[
            pl.BlockSpec((BE, HD), lambda i: (i, 0)),
            pl.BlockSpec((BE, H), lambda i: (i, 0)),
            pl.BlockSpec((1, 1, H), lambda i: (i, 0, 0)),
            pl.BlockSpec((1, 1, H), lambda i: (i, 0, 0)),
        ],
        out_shape=[
            jax.ShapeDtypeStruct((E, HD), jnp.bfloat16),
            jax.ShapeDtypeStruct((E, H), jnp.float32),
            jax.ShapeDtypeStruct((nb, 1, H), jnp.float32),
            jax.ShapeDtypeStruct((nb, 1, H), jnp.float32),
        ],
    )(gxi, gxj, ea, Wb, af, onesb)


def _tc_bn_exp(alpha, params):
    E, H = alpha.shape
    BC = 10000
    nb = E // BC

    def body(al_ref, p_ref, ex_ref):
        al = al_ref[...]
        mean = p_ref[0:1, :]
        rstdg = p_ref[1:2, :]
        beta = p_ref[2:3, :]
        z = (al - mean) * rstdg + beta
        ex = jnp.exp(_sp(z))
        ex_ref[...] = jnp.concatenate(
            [ex, jnp.zeros((BC, 16 - H), jnp.float32)], axis=1)

    return pl.pallas_call(
        body,
        grid=(nb,),
        in_specs=[
            pl.BlockSpec((BC, H), lambda i: (i, 0)),
            pl.BlockSpec((8, H), lambda i: (0, 0)),
        ],
        out_specs=pl.BlockSpec((BC, 16), lambda i: (i, 0)),
        out_shape=jax.ShapeDtypeStruct((E, 16), jnp.float32),
    )(alpha, params)


def _tc_densum(den):
    NP = den.shape[1]
    BN = NP // 8

    def body(d_ref, o_ref):
        o_ref[...] = d_ref[0] + d_ref[1]

    return pl.pallas_call(
        body,
        grid=(8,),
        in_specs=[pl.BlockSpec((2, BN, 16), lambda i: (0, i, 0))],
        out_specs=pl.BlockSpec((BN, 16), lambda i: (i, 0)),
        out_shape=jax.ShapeDtypeStruct((NP, 16), jnp.float32),
    )(den)


def _tc_combine(oj, ex16, dg):
    E, HD = oj.shape
    H = 4
    DIM = HD // H
    nb = E // BE

    def body(oj_ref, ex_ref, d_ref, m_ref):
        w = (ex_ref[:, :H] / (d_ref[:, :H] + 1e-16) * 0.25).astype(jnp.bfloat16)
        oj = oj_ref[...]
        acc = oj[:, 0:DIM] * w[:, 0:1]
        for h in range(1, H):
            acc = acc + oj[:, h * DIM:(h + 1) * DIM] * w[:, h:h + 1]
        m_ref[...] = acc.astype(jnp.float32)

    return pl.pallas_call(
        body,
        grid=(nb,),
        in_specs=[
            pl.BlockSpec((BE, HD), lambda i: (i, 0)),
            pl.BlockSpec((BE, 16), lambda i: (i, 0)),
            pl.BlockSpec((BE, 16), lambda i: (i, 0)),
        ],
        out_specs=pl.BlockSpec((BE, DIM), lambda i: (i, 0)),
        out_shape=jax.ShapeDtypeStruct((E, DIM), jnp.float32),
    )(oj, ex16, dg)


def _tc_finalize(p0, p1, biasb):
    N, DIM = p0.shape
    BN = 1000
    nb = N // BN

    def body(a_ref, b_ref, bias_ref, o_ref):
        o_ref[...] = a_ref[...] + b_ref[...] + bias_ref[0:1, :]

    return pl.pallas_call(
        body,
        grid=(nb,),
        in_specs=[
            pl.BlockSpec((BN, DIM), lambda i: (i, 0)),
            pl.BlockSpec((BN, DIM), lambda i: (i, 0)),
            pl.BlockSpec((8, DIM), lambda i: (0, 0)),
        ],
        out_specs=pl.BlockSpec((BN, DIM), lambda i: (i, 0)),
        out_shape=jax.ShapeDtypeStruct((N, DIM), jnp.float32),
    )(p0, p1, biasb)


def kernel(x, edge_index, edge_attr, W, att, bias, bn_gamma, bn_beta):
    N, DIM = x.shape
    E = edge_attr.shape[0]
    H = att.shape[1]

    idx_i = edge_index[0]
    idx_j = edge_index[1]
    eye = jnp.eye(H, dtype=jnp.float32)
    af = jnp.zeros((8, H * DIM), jnp.float32)
    af = af.at[0].set(att[0, :, :DIM].reshape(-1)).at[1].set(att[0, :, DIM:].reshape(-1))
    af = af.astype(jnp.bfloat16)
    onesb = (jnp.ones((H, DIM, 1)) * eye[:, None, :]).reshape(H * DIM, H).astype(jnp.bfloat16)
    Wb = W.astype(jnp.bfloat16)

    gxi, gxj = _sc_gather_x(x, idx_i, idx_j)
    oj, alpha, s_part, ss_part = _tc_edge(gxi, gxj, edge_attr, Wb, af, onesb)

    s = s_part.sum(axis=(0, 1))
    ss = ss_part.sum(axis=(0, 1))
    mean = s / E
    var = ss / E - mean * mean
    rstdg = bn_gamma / jnp.sqrt(var + 1e-5)
    params = jnp.zeros((8, H), jnp.float32)
    params = params.at[0].set(mean).at[1].set(rstdg).at[2].set(bn_beta)

    ex16 = _tc_bn_exp(alpha, params)
    den = _sc_scatter_add(ex16, idx_i, N, 16)
    dg = _sc_denom_gather(_tc_densum(den), idx_i)
    msgr = _tc_combine(oj, ex16, dg)
    agg = _sc_scatter_add(msgr, idx_i, N, 128)
    biasb = jnp.broadcast_to(bias, (8, DIM))
    return _tc_finalize(agg[0, :N], agg[1, :N], biasb)


# speedup vs baseline: 24.4425x; 1.0337x over previous
import functools

import jax
import jax.numpy as jnp
from jax import lax
from jax.experimental import pallas as pl
from jax.experimental.pallas import tpu as pltpu
from jax.experimental.pallas import tpu_sc as plsc

NC = 2
NS = 16
NW = NC * NS
CH = 128

BE = 4000


def _sp(x):
    return jnp.maximum(x, 0.0) + jnp.log1p(jnp.exp(-jnp.abs(x)))


def _sc_gather_x(x, idx_i, idx_j):
    E = idx_i.shape[0]
    N, DIM = x.shape
    ept = E // NW
    nfull = ept // CH
    tail = ept - nfull * CH
    mesh = plsc.VectorSubcoreMesh(core_axis_name="c", subcore_axis_name="s")

    @functools.partial(
        pl.kernel, mesh=mesh,
        out_type=(jax.ShapeDtypeStruct((E, DIM), x.dtype),
                  jax.ShapeDtypeStruct((E, DIM), x.dtype)),
        scratch_types=[pltpu.VMEM((ept,), jnp.int32),
                       pltpu.VMEM((ept,), jnp.int32),
                       pltpu.VMEM((CH, DIM), x.dtype),
                       pltpu.VMEM((CH, DIM), x.dtype),
                       pltpu.SemaphoreType.DMA,
                       pltpu.SemaphoreType.DMA,
                       pltpu.SemaphoreType.DMA,
                       pltpu.SemaphoreType.DMA],
    )
    def k(x_hbm, ii_hbm, ij_hbm, gi_hbm, gj_hbm,
          idxi_v, idxj_v, buf0, buf1, g0, g1, w0, w1):
        wid = lax.axis_index("s") * NC + lax.axis_index("c")
        base = wid * ept
        pltpu.sync_copy(ii_hbm.at[pl.ds(base, ept)], idxi_v)
        pltpu.sync_copy(ij_hbm.at[pl.ds(base, ept)], idxj_v)
        bufs = (buf0, buf1)
        gsem = (g0, g1)
        wsem = (w0, w1)

        for idx_v, out_hbm in ((idxi_v, gi_hbm), (idxj_v, gj_hbm)):
            def gat(cur, b):
                return pltpu.make_async_copy(
                    x_hbm.at[idx_v.at[pl.ds(cur * CH, CH)]], bufs[b], gsem[b])

            def wrb(cur, b):
                return pltpu.make_async_copy(
                    bufs[b], out_hbm.at[pl.ds(base + cur * CH, CH)], wsem[b])

            for b in range(2):
                gat(b, b).start()

            @pl.loop(0, nfull, step=2)
            def _(t):
                for b in range(2):
                    cur = t + b
                    gat(cur, b).wait()
                    wrb(cur, b).start()
                    wrb(cur, b).wait()

                    @pl.when(cur + 2 < nfull)
                    def _():
                        gat(cur + 2, b).start()

            if tail:
                pltpu.sync_copy(x_hbm.at[idx_v.at[pl.ds(nfull * CH, tail)]],
                                buf0.at[pl.ds(0, tail)])
                pltpu.sync_copy(buf0.at[pl.ds(0, tail)],
                                out_hbm.at[pl.ds(base + nfull * CH, tail)])

    return k(x, idx_i, idx_j)


def _sc_scatter_add(vals, idx_i, n_nodes, width):
    E = vals.shape[0]
    nchunks = E // CH
    per_core = nchunks // NC
    per_tile = -(-per_core // NS)
    n_pad = -(-n_nodes // (8 * NS)) * 8 * NS
    rows_per_tile = n_pad // NS
    mesh = plsc.VectorSubcoreMesh(core_axis_name="c", subcore_axis_name="s")
    zrows = jnp.zeros((rows_per_tile, width), jnp.float32)

    @functools.partial(
        pl.kernel, mesh=mesh,
        out_type=jax.ShapeDtypeStruct((NC, n_pad, width), jnp.float32),
        scratch_types=[pltpu.VMEM((CH,), jnp.int32),
                       pltpu.VMEM((CH,), jnp.int32),
                       pltpu.VMEM((CH, width), jnp.float32),
                       pltpu.VMEM((CH, width), jnp.float32),
                       pltpu.VMEM_SHARED((n_pad, width), jnp.float32),
                       pltpu.SemaphoreType.DMA,
                       pltpu.SemaphoreType.DMA,
                       pltpu.SemaphoreType.DMA,
                       pltpu.SemaphoreType.DMA,
                       pltpu.SemaphoreType.DMA,
                       pltpu.SemaphoreType.DMA],
        compiler_params=(pltpu.CompilerParams(use_tc_tiling_on_sc=False)
                         if width < 128 else None),
    )
    def k(v_hbm, ii_hbm, z_hbm, out_hbm, idx0, idx1, buf0, buf1, acc_shared,
          i0, i1, g0, g1, w0, w1):
        c = lax.axis_index("c")
        s = lax.axis_index("s")
        pltpu.sync_copy(z_hbm, acc_shared.at[pl.ds(s * rows_per_tile, rows_per_tile)])
        plsc.subcore_barrier()
        idxs = (idx0, idx1)
        bufs = (buf0, buf1)
        isem = (i0, i1)
        gsem = (g0, g1)
        wsem = (w0, w1)

        def chunk_of(t):
            return c * per_core + s + NS * t

        def load(t, b):
            base = chunk_of(t) * CH
            pltpu.make_async_copy(ii_hbm.at[pl.ds(base, CH)], idxs[b],
                                  isem[b]).start()
            pltpu.make_async_copy(v_hbm.at[pl.ds(base, CH)], bufs[b],
                                  gsem[b]).start()

        def load_wait(t, b):
            base = chunk_of(t) * CH
            pltpu.make_async_copy(ii_hbm.at[pl.ds(base, CH)], idxs[b],
                                  isem[b]).wait()
            pltpu.make_async_copy(v_hbm.at[pl.ds(base, CH)], bufs[b],
                                  gsem[b]).wait()

        for b in range(2):
            @pl.when(s + NS * b < per_core)
            def _():
                load(b, b)

        @pl.loop(0, per_tile, step=2)
        def _(t):
            for b in range(2):
                tt = t + b
                kk = s + NS * tt

                @pl.when(kk < per_core)
                def _():
                    load_wait(tt, b)
                    pltpu.async_copy(bufs[b], acc_shared.at[idxs[b]],
                                     wsem[b], add=True)
                    pltpu.make_async_copy(bufs[b], acc_shared.at[idxs[b]],
                                          wsem[b]).wait()

                    @pl.when(s + NS * (tt + 2) < per_core)
                    def _():
                        load(tt + 2, b)

        plsc.subcore_barrier()
        sl = pl.ds(s * rows_per_tile, rows_per_tile)
        pltpu.sync_copy(acc_shared.at[sl], out_hbm.at[c].at[sl])

    return k(vals, idx_i, zrows)


def _sc_denom_gather(den, idx_i):
    E = idx_i.shape[0]
    ept = E // NW
    nfull = ept // CH
    tail = ept - nfull * CH
    mesh = plsc.VectorSubcoreMesh(core_axis_name="c", subcore_axis_name="s")

    @functools.partial(
        pl.kernel, mesh=mesh,
        out_type=jax.ShapeDtypeStruct((E, 16), jnp.float32),
        scratch_types=[pltpu.VMEM((ept,), jnp.int32),
                       pltpu.VMEM((CH, 16), jnp.float32),
                       pltpu.VMEM((CH, 16), jnp.float32),
                       pltpu.SemaphoreType.DMA,
                       pltpu.SemaphoreType.DMA,
                       pltpu.SemaphoreType.DMA,
                       pltpu.SemaphoreType.DMA],
        compiler_params=pltpu.CompilerParams(use_tc_tiling_on_sc=False),
    )
    def k(den_hbm, ii_hbm, dg_hbm, idx_v, buf0, buf1, g0, g1, w0, w1):
        wid = lax.axis_index("s") * NC + lax.axis_index("c")
        base = wid * ept
        pltpu.sync_copy(ii_hbm.at[pl.ds(base, ept)], idx_v)
        bufs = (buf0, buf1)
        gsem = (g0, g1)
        wsem = (w0, w1)

        def gat(cur, b):
            return pltpu.make_async_copy(
                den_hbm.at[idx_v.at[pl.ds(cur * CH, CH)]], bufs[b], gsem[b])

        def wrb(cur, b):
            return pltpu.make_async_copy(
                bufs[b], dg_hbm.at[pl.ds(base + cur * CH, CH)], wsem[b])

        for b in range(2):
            gat(b, b).start()

        @pl.loop(0, nfull, step=2)
        def _(t):
            for b in range(2):
                cur = t + b
                gat(cur, b).wait()
                wrb(cur, b).start()
                wrb(cur, b).wait()

                @pl.when(cur + 2 < nfull)
                def _():
                    gat(cur + 2, b).start()

        if tail:
            pltpu.sync_copy(den_hbm.at[idx_v.at[pl.ds(nfull * CH, tail)]],
                            buf0.at[pl.ds(0, tail)])
            pltpu.sync_copy(buf0.at[pl.ds(0, tail)],
                            dg_hbm.at[pl.ds(base + nfull * CH, tail)])

    return k(den, idx_i)


def _tc_edge(gxi, gxj, ea, Wb, af, onesb):
    E, DIM = ea.shape
    HD = Wb.shape[1]
    H = onesb.shape[1]
    nb = E // BE

    def body(gxi_ref, gxj_ref, ea_ref, w_ref, af_ref, ones_ref,
             oj_ref, al_ref, s_ref, ss_ref):
        eab = ea_ref[...].astype(jnp.bfloat16)
        ci = jnp.concatenate([gxi_ref[...].astype(jnp.bfloat16), eab], axis=1)
        cj = jnp.concatenate([gxj_ref[...].astype(jnp.bfloat16), eab], axis=1)
        w = w_ref[...]
        ui = jnp.dot(ci, w, preferred_element_type=jnp.float32).astype(jnp.bfloat16)
        uj = jnp.dot(cj, w, preferred_element_type=jnp.float32).astype(jnp.bfloat16)
        oi = _sp(ui)
        oj = _sp(uj)
        oj_ref[...] = oj
        v = oi * af_ref[0:1, :] + oj * af_ref[1:2, :]
        al = jnp.dot(v, ones_ref[...], preferred_element_type=jnp.float32)
        al = _sp(al)
        al_ref[...] = al
        s_ref[...] = al.sum(axis=0).reshape(1, 1, H)
        ss_ref[...] = (al * al).sum(axis=0).reshape(1, 1, H)

    return pl.pallas_call(
        body,
        grid=(nb,),
        in_specs=[
            pl.BlockSpec((BE, DIM), lambda i: (i, 0)),
            pl.BlockSpec((BE, DIM), lambda i: (i, 0)),
            pl.BlockSpec((BE, DIM), lambda i: (i, 0)),
            pl.BlockSpec((2 * DIM, HD), lambda i: (0, 0)),
            pl.BlockSpec((8, HD), lambda i: (0, 0)),
            pl.BlockSpec((HD, H), lambda i: (0, 0)),
        ],
        out_specs=[
            pl.BlockSpec((BE, HD), lambda i: (i, 0)),
            pl.BlockSpec((BE, H), lambda i: (i, 0)),
            pl.BlockSpec((1, 1, H), lambda i: (i, 0, 0)),
            pl.BlockSpec((1, 1, H), lambda i: (i, 0, 0)),
        ],
        out_shape=[
            jax.ShapeDtypeStruct((E, HD), jnp.bfloat16),
            jax.ShapeDtypeStruct((E, H), jnp.float32),
            jax.ShapeDtypeStruct((nb, 1, H), jnp.float32),
            jax.ShapeDtypeStruct((nb, 1, H), jnp.float32),
        ],
    )(gxi, gxj, ea, Wb, af, onesb)


def _tc_bn_exp(alpha, params):
    E, H = alpha.shape
    BC = 10000
    nb = E // BC

    def body(al_ref, p_ref, ex_ref):
        al = al_ref[...]
        mean = p_ref[0:1, :]
        rstdg = p_ref[1:2, :]
        beta = p_ref[2:3, :]
        z = (al - mean) * rstdg + beta
        ex = jnp.exp(_sp(z))
        ex_ref[...] = jnp.concatenate(
            [ex, jnp.zeros((BC, 16 - H), jnp.float32)], axis=1)

    return pl.pallas_call(
        body,
        grid=(nb,),
        in_specs=[
            pl.BlockSpec((BC, H), lambda i: (i, 0)),
            pl.BlockSpec((8, H), lambda i: (0, 0)),
        ],
        out_specs=pl.BlockSpec((BC, 16), lambda i: (i, 0)),
        out_shape=jax.ShapeDtypeStruct((E, 16), jnp.float32),
    )(alpha, params)


def _tc_densum(den):
    NP = den.shape[1]
    BN = NP // 8

    def body(d_ref, o_ref):
        o_ref[...] = d_ref[0] + d_ref[1]

    return pl.pallas_call(
        body,
        grid=(8,),
        in_specs=[pl.BlockSpec((2, BN, 16), lambda i: (0, i, 0))],
        out_specs=pl.BlockSpec((BN, 16), lambda i: (i, 0)),
        out_shape=jax.ShapeDtypeStruct((NP, 16), jnp.float32),
    )(den)


def _tc_combine(oj, ex16, dg):
    E, HD = oj.shape
    H = 4
    DIM = HD // H
    nb = E // BE

    def body(oj_ref, ex_ref, d_ref, m_ref):
        w = (ex_ref[:, :H] / (d_ref[:, :H] + 1e-16) * 0.25).astype(jnp.bfloat16)
        oj = oj_ref[...]
        acc = oj[:, 0:DIM] * w[:, 0:1]
        for h in range(1, H):
            acc = acc + oj[:, h * DIM:(h + 1) * DIM] * w[:, h:h + 1]
        m_ref[...] = acc.astype(jnp.float32)

    return pl.pallas_call(
        body,
        grid=(nb,),
        in_specs=[
            pl.BlockSpec((BE, HD), lambda i: (i, 0)),
            pl.BlockSpec((BE, 16), lambda i: (i, 0)),
            pl.BlockSpec((BE, 16), lambda i: (i, 0)),
        ],
        out_specs=pl.BlockSpec((BE, DIM), lambda i: (i, 0)),
        out_shape=jax.ShapeDtypeStruct((E, DIM), jnp.float32),
    )(oj, ex16, dg)


def _tc_finalize(p0, p1, biasb):
    N, DIM = p0.shape
    BN = 1000
    nb = N // BN

    def body(a_ref, b_ref, bias_ref, o_ref):
        o_ref[...] = a_ref[...] + b_ref[...] + bias_ref[0:1, :]

    return pl.pallas_call(
        body,
        grid=(nb,),
        in_specs=[
            pl.BlockSpec((BN, DIM), lambda i: (i, 0)),
            pl.BlockSpec((BN, DIM), lambda i: (i, 0)),
            pl.BlockSpec((8, DIM), lambda i: (0, 0)),
        ],
        out_specs=pl.BlockSpec((BN, DIM), lambda i: (i, 0)),
        out_shape=jax.ShapeDtypeStruct((N, DIM), jnp.float32),
    )(p0, p1, biasb)


def kernel(x, edge_index, edge_attr, W, att, bias, bn_gamma, bn_beta):
    N, DIM = x.shape
    E = edge_attr.shape[0]
    H = att.shape[1]

    idx_i = edge_index[0]
    idx_j = edge_index[1]
    eye = jnp.eye(H, dtype=jnp.float32)
    af = jnp.zeros((8, H * DIM), jnp.float32)
    af = af.at[0].set(att[0, :, :DIM].reshape(-1)).at[1].set(att[0, :, DIM:].reshape(-1))
    af = af.astype(jnp.bfloat16)
    onesb = (jnp.ones((H, DIM, 1)) * eye[:, None, :]).reshape(H * DIM, H).astype(jnp.bfloat16)
    Wb = W.astype(jnp.bfloat16)

    gxi, gxj = _sc_gather_x(x, idx_i, idx_j)
    oj, alpha, s_part, ss_part = _tc_edge(gxi, gxj, edge_attr, Wb, af, onesb)

    s = s_part.sum(axis=(0, 1))
    ss = ss_part.sum(axis=(0, 1))
    mean = s / E
    var = ss / E - mean * mean
    rstdg = bn_gamma / jnp.sqrt(var + 1e-5)
    params = jnp.zeros((8, H), jnp.float32)
    params = params.at[0].set(mean).at[1].set(rstdg).at[2].set(bn_beta)

    ex16 = _tc_bn_exp(alpha, params)
    den = _sc_scatter_add(ex16, idx_i, N, 16)
    dg = _sc_denom_gather(_tc_densum(den), idx_i)
    msgr = _tc_combine(oj, ex16, dg)
    agg = _sc_scatter_add(msgr, idx_i, N, 128)
    biasb = jnp.broadcast_to(bias, (8, DIM))
    return _tc_finalize(agg[0, :N], agg[1, :N], biasb)
